# baseline (speedup 1.0000x reference)
import functools

import jax
import jax.numpy as jnp
from jax import lax
from jax.experimental import pallas as pl
from jax.experimental.pallas import tpu as pltpu
from jax.experimental.pallas import tpu_sc as plsc

N_NODES = 10000
N_EDGES = 320000
HIDDEN = 128
MSG_DIM = 16
BATCH = 2048
T_MAX = 1000.0

NC, NS = 2, 16
NW = NC * NS
E_PER_W = N_EDGES // NW
CHUNK = 80
N_CHUNKS = E_PER_W // CHUNK
ZROW = N_NODES
N_PAD = N_NODES + 8
B_PER_TILE = BATCH // NS
TRASH = BATCH
N_ACC = 2176
ROWS_PER_TILE = N_ACC // NS

BE = 3200


def _edge_feat_body(t_ref, src_ref, msg_ref, w_ref, wm_ref, bm_ref,
                    f_ref, srcm_ref):
    t = t_ref[...]
    mask = t <= T_MAX
    f = (jnp.cos(t * w_ref[...])
         + jnp.dot(msg_ref[...], wm_ref[...],
                   preferred_element_type=jnp.float32)
         + bm_ref[...])
    f_ref[...] = jnp.where(mask, f, 0.0)
    srcm_ref[...] = jnp.where(mask, src_ref[...], ZROW)


def _edge_features(t2, src2, msg, w_time, W_msg, bm):
    grid = N_EDGES // BE
    return pl.pallas_call(
        _edge_feat_body,
        grid=(grid,),
        in_specs=[
            pl.BlockSpec((BE, 1), lambda g: (g, 0)),
            pl.BlockSpec((BE, 1), lambda g: (g, 0)),
            pl.BlockSpec((BE, MSG_DIM), lambda g: (g, 0)),
            pl.BlockSpec((1, HIDDEN), lambda g: (0, 0)),
            pl.BlockSpec((MSG_DIM, HIDDEN), lambda g: (0, 0)),
            pl.BlockSpec((1, HIDDEN), lambda g: (0, 0)),
        ],
        out_specs=[
            pl.BlockSpec((BE, HIDDEN), lambda g: (g, 0)),
            pl.BlockSpec((BE, 1), lambda g: (g, 0)),
        ],
        out_shape=[
            jax.ShapeDtypeStruct((N_EDGES, HIDDEN), jnp.float32),
            jax.ShapeDtypeStruct((N_EDGES, 1), jnp.int32),
        ],
    )(t2, src2, msg, w_time, W_msg, bm)


def _sc_agg_body(emb_hbm, f_hbm, srcm_hbm, dst_hbm, idx_hbm, zeros_hbm,
                 inv_hbm, sel_out, acc, slotmap, srcv, dstv, idxall, slotsel,
                 rows0, fbuf0, rows1, fbuf1, selbuf,
                 gsem0, fsem0, gsem1, fsem1, ssem):
    c = lax.axis_index("c")
    s = lax.axis_index("s")
    wid = c * NS + s
    pltpu.sync_copy(zeros_hbm.at[pl.ds(s * ROWS_PER_TILE, ROWS_PER_TILE), :],
                    acc.at[pl.ds(s * ROWS_PER_TILE, ROWS_PER_TILE), :])
    pltpu.sync_copy(inv_hbm, slotmap)
    pltpu.sync_copy(idx_hbm, idxall)
    lanes = jnp.arange(16, dtype=jnp.int32)

    def sbody(k, carry):
        vi = idxall[pl.ds(k * 16, 16)]
        plsc.store_scatter(slotmap, [vi], lanes + k * 16)
        return carry

    lax.fori_loop(0, BATCH // 16, sbody, 0)
    pltpu.sync_copy(srcm_hbm.at[wid], srcv)
    pltpu.sync_copy(dst_hbm.at[wid], dstv)

    def mbody(i, carry):
        r = i // (CHUNK // 16)
        col = (i % (CHUNK // 16)) * 16
        d = dstv[r, pl.ds(col, 16)]
        dstv[r, pl.ds(col, 16)] = plsc.load_gather(slotmap, [d])
        return carry

    lax.fori_loop(0, E_PER_W // 16, mbody, 0)
    plsc.subcore_barrier()

    base_e = wid * E_PER_W

    def start_loads(k, rows, fbuf, gsem, fsem):
        pltpu.async_copy(emb_hbm.at[srcv.at[k]], rows, gsem)
        pltpu.async_copy(f_hbm.at[pl.ds(base_e + k * CHUNK, CHUNK), :],
                         fbuf, fsem)

    def wait_loads(k, rows, fbuf, gsem, fsem):
        pltpu.make_async_copy(emb_hbm.at[srcv.at[k]], rows, gsem).wait()
        pltpu.make_async_copy(f_hbm.at[pl.ds(base_e + k * CHUNK, CHUNK), :],
                              fbuf, fsem).wait()

    def scatter(k, rows, fbuf):
        cp = pltpu.async_copy(rows, acc.at[dstv.at[k]], ssem, add=True)
        pltpu.sync_copy(fbuf, acc.at[dstv.at[k]], add=True)
        cp.wait()

    start_loads(0, rows0, fbuf0, gsem0, fsem0)

    def body(j, carry):
        a = 2 * j
        start_loads(a + 1, rows1, fbuf1, gsem1, fsem1)
        wait_loads(a, rows0, fbuf0, gsem0, fsem0)
        scatter(a, rows0, fbuf0)
        start_loads(a + 2, rows0, fbuf0, gsem0, fsem0)
        wait_loads(a + 1, rows1, fbuf1, gsem1, fsem1)
        scatter(a + 1, rows1, fbuf1)
        return carry

    lax.fori_loop(0, (N_CHUNKS - 1) // 2, body, 0)
    wait_loads(N_CHUNKS - 1, rows0, fbuf0, gsem0, fsem0)
    scatter(N_CHUNKS - 1, rows0, fbuf0)
    plsc.subcore_barrier()

    for k in range(B_PER_TILE // 16):
        vi = idxall[pl.ds(s * B_PER_TILE + k * 16, 16)]
        slotsel[pl.ds(k * 16, 16)] = plsc.load_gather(slotmap, [vi])
    pltpu.sync_copy(acc.at[slotsel], selbuf)
    pltpu.sync_copy(selbuf, sel_out.at[c, pl.ds(s * B_PER_TILE, B_PER_TILE), :])

    @pl.when(c == 0)
    def _():
        pltpu.async_copy(
            emb_hbm.at[idxall.at[pl.ds(s * B_PER_TILE, B_PER_TILE)]],
            selbuf, gsem0).wait()
        pltpu.sync_copy(selbuf,
                        sel_out.at[2, pl.ds(s * B_PER_TILE, B_PER_TILE), :])


@functools.cache
def _make_sc_agg():
    return functools.partial(
        pl.kernel,
        out_type=jax.ShapeDtypeStruct((3, BATCH, HIDDEN), jnp.float32),
        mesh=plsc.VectorSubcoreMesh(core_axis_name="c", subcore_axis_name="s",
                                    num_cores=NC, num_subcores=NS),
        compiler_params=pltpu.CompilerParams(use_tc_tiling_on_sc=False,
                                             needs_layout_passes=False),
        scratch_types=[
            pltpu.VMEM_SHARED((N_ACC, HIDDEN), jnp.float32),
            pltpu.VMEM((N_NODES,), jnp.int32),
            pltpu.VMEM((N_CHUNKS, CHUNK), jnp.int32),
            pltpu.VMEM((N_CHUNKS, CHUNK), jnp.int32),
            pltpu.VMEM((BATCH,), jnp.int32),
            pltpu.VMEM((B_PER_TILE,), jnp.int32),
            pltpu.VMEM((CHUNK, HIDDEN), jnp.float32),
            pltpu.VMEM((CHUNK, HIDDEN), jnp.float32),
            pltpu.VMEM((CHUNK, HIDDEN), jnp.float32),
            pltpu.VMEM((CHUNK, HIDDEN), jnp.float32),
            pltpu.VMEM((B_PER_TILE, HIDDEN), jnp.float32),
            pltpu.SemaphoreType.DMA,
            pltpu.SemaphoreType.DMA,
            pltpu.SemaphoreType.DMA,
            pltpu.SemaphoreType.DMA,
            pltpu.SemaphoreType.DMA,
        ],
    )(_sc_agg_body)


def _cls_body(sel_ref, wu_ref, bu_ref, w1_ref, b1_ref, w2_ref, b2_ref,
              out_ref):
    x = sel_ref[0] + sel_ref[1] + sel_ref[2]
    h = jnp.maximum(
        jnp.dot(x, wu_ref[...], preferred_element_type=jnp.float32)
        + bu_ref[...], 0.0)
    h2 = jnp.maximum(
        jnp.dot(h, w1_ref[...], preferred_element_type=jnp.float32)
        + b1_ref[...], 0.0)
    out_ref[...] = (jnp.dot(h2, w2_ref[...],
                            preferred_element_type=jnp.float32)
                    + b2_ref[...])


def _classifier(sel, W_upd, bu, W1, b1, W2, b2):
    return pl.pallas_call(
        _cls_body,
        out_shape=jax.ShapeDtypeStruct((BATCH, HIDDEN), jnp.float32),
    )(sel, W_upd, bu, W1, b1, W2, b2)


def kernel(src, dst, t, msg, labels, idx, node_emb, w_time, W_msg, b_msg,
           W_upd, b_upd, W1, b1, W2, b2):
    del labels
    t2 = t.reshape(N_EDGES, 1)
    src2 = src.reshape(N_EDGES, 1)
    F, srcm = _edge_features(t2, src2, msg, w_time, W_msg,
                             b_msg.reshape(1, HIDDEN))
    emb_pad = jnp.concatenate(
        [node_emb, jnp.zeros((N_PAD - N_NODES, HIDDEN), node_emb.dtype)],
        axis=0)
    sel = _make_sc_agg()(emb_pad, F,
                  srcm.reshape(NW, N_CHUNKS, CHUNK),
                  dst.reshape(NW, N_CHUNKS, CHUNK),
                  idx,
                  jnp.zeros((N_ACC, HIDDEN), jnp.float32),
                  jnp.full((N_NODES,), TRASH, jnp.int32))
    return _classifier(sel, W_upd, b_upd.reshape(1, HIDDEN), W1,
                       b1.reshape(1, HIDDEN), W2, b2.reshape(1, HIDDEN))


# speedup vs baseline: 2.7650x; 2.7650x over previous
import functools

import jax
import jax.numpy as jnp
from jax import lax
from jax.experimental import pallas as pl
from jax.experimental.pallas import tpu as pltpu
from jax.experimental.pallas import tpu_sc as plsc

N_NODES = 10000
N_EDGES = 320000
HIDDEN = 128
MSG_DIM = 16
BATCH = 2048
T_MAX = 1000.0

NC, NS = 2, 16
NW = NC * NS
E_PER_W = N_EDGES // NW
CHUNK = 80
N_CHUNKS = E_PER_W // CHUNK
ZROW = N_NODES
N_PAD = N_NODES + 8
B_PER_TILE = BATCH // NS
TRASH = BATCH
N_ACC = 2176
ROWS_PER_TILE = N_ACC // NS

BE = 3200


def _edge_feat_body(t_ref, src_ref, msg_ref, w_ref, wm_ref, bm_ref,
                    f_ref, srcm_ref):
    t = t_ref[...]
    mask = t <= T_MAX
    f = (jnp.cos(t * w_ref[...])
         + jnp.dot(msg_ref[...], wm_ref[...],
                   preferred_element_type=jnp.float32)
         + bm_ref[...])
    f_ref[...] = jnp.where(mask, f, 0.0)
    srcm_ref[...] = jnp.where(mask, src_ref[...], ZROW)


def _edge_features(t2, src2, msg, w_time, W_msg, bm):
    grid = N_EDGES // BE
    return pl.pallas_call(
        _edge_feat_body,
        grid=(grid,),
        in_specs=[
            pl.BlockSpec((BE, 1), lambda g: (g, 0)),
            pl.BlockSpec((BE, 1), lambda g: (g, 0)),
            pl.BlockSpec((BE, MSG_DIM), lambda g: (g, 0)),
            pl.BlockSpec((1, HIDDEN), lambda g: (0, 0)),
            pl.BlockSpec((MSG_DIM, HIDDEN), lambda g: (0, 0)),
            pl.BlockSpec((1, HIDDEN), lambda g: (0, 0)),
        ],
        out_specs=[
            pl.BlockSpec((BE, HIDDEN), lambda g: (g, 0)),
            pl.BlockSpec((BE, 1), lambda g: (g, 0)),
        ],
        out_shape=[
            jax.ShapeDtypeStruct((N_EDGES, HIDDEN), jnp.float32),
            jax.ShapeDtypeStruct((N_EDGES, 1), jnp.int32),
        ],
    )(t2, src2, msg, w_time, W_msg, bm)


def _sc_agg_body(emb_hbm, f_hbm, srcm_hbm, dst_hbm, idx_hbm, zeros_hbm,
                 inv_hbm, sel_out, acc, slotmap, srcv, dstv, eidc, dstb,
                 idxall, slotsel, rows0, fbuf0, rows1, fbuf1, selbuf,
                 gsem0, fsem0, gsem1, fsem1, ssem):
    c = lax.axis_index("c")
    s = lax.axis_index("s")
    wid = c * NS + s
    pltpu.sync_copy(zeros_hbm.at[pl.ds(s * ROWS_PER_TILE, ROWS_PER_TILE), :],
                    acc.at[pl.ds(s * ROWS_PER_TILE, ROWS_PER_TILE), :])
    pltpu.sync_copy(inv_hbm, slotmap)
    pltpu.sync_copy(idx_hbm, idxall)
    lanes = jnp.arange(16, dtype=jnp.int32)

    def sbody(k, carry):
        vi = idxall[pl.ds(k * 16, 16)]
        plsc.store_scatter(slotmap, [vi], lanes + k * 16)
        return carry

    lax.fori_loop(0, BATCH // 16, sbody, 0)
    pltpu.sync_copy(srcm_hbm.at[wid], srcv.at[pl.ds(0, E_PER_W)])
    pltpu.sync_copy(dst_hbm.at[wid], dstv.at[pl.ds(0, E_PER_W)])

    base_e = wid * E_PER_W

    def cbody(i, off):
        d = dstv[pl.ds(i * 16, 16)]
        sv = srcv[pl.ds(i * 16, 16)]
        slot = plsc.load_gather(slotmap, [d])
        m = slot != TRASH
        plsc.store_compressed(dstv.at[pl.ds(off, 16)], slot, mask=m)
        plsc.store_compressed(srcv.at[pl.ds(off, 16)], sv, mask=m)
        plsc.store_compressed(eidc.at[pl.ds(off, 16)],
                              lanes + (base_e + i * 16), mask=m)
        return off + jnp.sum(m.astype(jnp.int32))

    cnt = lax.fori_loop(0, E_PER_W // 16, cbody, jnp.int32(0))

    def pbody(g, carry):
        dstv[pl.ds(cnt + g * 16, 16)] = jnp.full((16,), TRASH, jnp.int32)
        srcv[pl.ds(cnt + g * 16, 16)] = jnp.full((16,), ZROW, jnp.int32)
        eidc[pl.ds(cnt + g * 16, 16)] = jnp.zeros((16,), jnp.int32)
        return carry

    lax.fori_loop(0, 2 * CHUNK // 16, pbody, 0)
    plsc.subcore_barrier()

    def start_loads(k, rows, fbuf, gsem, fsem):
        pltpu.async_copy(emb_hbm.at[srcv.at[pl.ds(k * CHUNK, CHUNK)]],
                         rows, gsem)
        pltpu.async_copy(f_hbm.at[eidc.at[pl.ds(k * CHUNK, CHUNK)]],
                         fbuf, fsem)

    def wait_loads(k, rows, fbuf, gsem, fsem):
        pltpu.make_async_copy(emb_hbm.at[srcv.at[pl.ds(k * CHUNK, CHUNK)]],
                              rows, gsem).wait()
        pltpu.make_async_copy(f_hbm.at[eidc.at[pl.ds(k * CHUNK, CHUNK)]],
                              fbuf, fsem).wait()

    def scatter(k, rows, fbuf):
        for v in range(CHUNK // 16):
            dstb[pl.ds(v * 16, 16)] = dstv[pl.ds(k * CHUNK + v * 16, 16)]
        cp = pltpu.async_copy(rows, acc.at[dstb], ssem, add=True)
        pltpu.sync_copy(fbuf, acc.at[dstb], add=True)
        cp.wait()

    npairs = (cnt + CHUNK - 1) // CHUNK // 2

    start_loads(0, rows0, fbuf0, gsem0, fsem0)

    def body(j, carry):
        a = 2 * j
        start_loads(a + 1, rows1, fbuf1, gsem1, fsem1)
        wait_loads(a, rows0, fbuf0, gsem0, fsem0)
        scatter(a, rows0, fbuf0)
        start_loads(a + 2, rows0, fbuf0, gsem0, fsem0)
        wait_loads(a + 1, rows1, fbuf1, gsem1, fsem1)
        scatter(a + 1, rows1, fbuf1)
        return carry

    lax.fori_loop(0, npairs, body, 0)
    wait_loads(2 * npairs, rows0, fbuf0, gsem0, fsem0)
    scatter(2 * npairs, rows0, fbuf0)
    plsc.subcore_barrier()

    for k in range(B_PER_TILE // 16):
        vi = idxall[pl.ds(s * B_PER_TILE + k * 16, 16)]
        slotsel[pl.ds(k * 16, 16)] = plsc.load_gather(slotmap, [vi])
    pltpu.sync_copy(acc.at[slotsel], selbuf)
    pltpu.sync_copy(selbuf, sel_out.at[c, pl.ds(s * B_PER_TILE, B_PER_TILE), :])

    @pl.when(c == 0)
    def _():
        pltpu.async_copy(
            emb_hbm.at[idxall.at[pl.ds(s * B_PER_TILE, B_PER_TILE)]],
            selbuf, gsem0).wait()
        pltpu.sync_copy(selbuf,
                        sel_out.at[2, pl.ds(s * B_PER_TILE, B_PER_TILE), :])


@functools.cache
def _make_sc_agg():
    return functools.partial(
        pl.kernel,
        out_type=jax.ShapeDtypeStruct((3, BATCH, HIDDEN), jnp.float32),
        mesh=plsc.VectorSubcoreMesh(core_axis_name="c", subcore_axis_name="s",
                                    num_cores=NC, num_subcores=NS),
        compiler_params=pltpu.CompilerParams(use_tc_tiling_on_sc=False,
                                             needs_layout_passes=False),
        scratch_types=[
            pltpu.VMEM_SHARED((N_ACC, HIDDEN), jnp.float32),
            pltpu.VMEM((N_NODES,), jnp.int32),
            pltpu.VMEM((E_PER_W + 2 * CHUNK,), jnp.int32),
            pltpu.VMEM((E_PER_W + 2 * CHUNK,), jnp.int32),
            pltpu.VMEM((E_PER_W + 2 * CHUNK,), jnp.int32),
            pltpu.VMEM((CHUNK,), jnp.int32),
            pltpu.VMEM((BATCH,), jnp.int32),
            pltpu.VMEM((B_PER_TILE,), jnp.int32),
            pltpu.VMEM((CHUNK, HIDDEN), jnp.float32),
            pltpu.VMEM((CHUNK, HIDDEN), jnp.float32),
            pltpu.VMEM((CHUNK, HIDDEN), jnp.float32),
            pltpu.VMEM((CHUNK, HIDDEN), jnp.float32),
            pltpu.VMEM((B_PER_TILE, HIDDEN), jnp.float32),
            pltpu.SemaphoreType.DMA,
            pltpu.SemaphoreType.DMA,
            pltpu.SemaphoreType.DMA,
            pltpu.SemaphoreType.DMA,
            pltpu.SemaphoreType.DMA,
        ],
    )(_sc_agg_body)


def _cls_body(sel_ref, wu_ref, bu_ref, w1_ref, b1_ref, w2_ref, b2_ref,
              out_ref):
    x = sel_ref[0] + sel_ref[1] + sel_ref[2]
    h = jnp.maximum(
        jnp.dot(x, wu_ref[...], preferred_element_type=jnp.float32)
        + bu_ref[...], 0.0)
    h2 = jnp.maximum(
        jnp.dot(h, w1_ref[...], preferred_element_type=jnp.float32)
        + b1_ref[...], 0.0)
    out_ref[...] = (jnp.dot(h2, w2_ref[...],
                            preferred_element_type=jnp.float32)
                    + b2_ref[...])


def _classifier(sel, W_upd, bu, W1, b1, W2, b2):
    return pl.pallas_call(
        _cls_body,
        out_shape=jax.ShapeDtypeStruct((BATCH, HIDDEN), jnp.float32),
    )(sel, W_upd, bu, W1, b1, W2, b2)


def kernel(src, dst, t, msg, labels, idx, node_emb, w_time, W_msg, b_msg,
           W_upd, b_upd, W1, b1, W2, b2):
    del labels
    t2 = t.reshape(N_EDGES, 1)
    src2 = src.reshape(N_EDGES, 1)
    F, srcm = _edge_features(t2, src2, msg, w_time, W_msg,
                             b_msg.reshape(1, HIDDEN))
    emb_pad = jnp.concatenate(
        [node_emb, jnp.zeros((N_PAD - N_NODES, HIDDEN), node_emb.dtype)],
        axis=0)
    sel = _make_sc_agg()(emb_pad, F,
                  srcm.reshape(NW, E_PER_W),
                  dst.reshape(NW, E_PER_W),
                  idx,
                  jnp.zeros((N_ACC, HIDDEN), jnp.float32),
                  jnp.full((N_NODES,), TRASH, jnp.int32))
    return _classifier(sel, W_upd, b_upd.reshape(1, HIDDEN), W1,
                       b1.reshape(1, HIDDEN), W2, b2.reshape(1, HIDDEN))


# speedup vs baseline: 3.0948x; 1.1193x over previous
import functools

import jax
import jax.numpy as jnp
from jax import lax
from jax.experimental import pallas as pl
from jax.experimental.pallas import tpu as pltpu
from jax.experimental.pallas import tpu_sc as plsc

N_NODES = 10000
N_EDGES = 320000
HIDDEN = 128
MSG_DIM = 16
BATCH = 2048
T_MAX = 1000.0

NC, NS = 2, 16
NW = NC * NS
E_PER_W = N_EDGES // NW
CHUNK = 80
N_CHUNKS = E_PER_W // CHUNK
ZROW = N_NODES
N_PAD = N_NODES + 8
B_PER_TILE = BATCH // NS
TRASH = BATCH
N_ACC = 2176
ROWS_PER_TILE = N_ACC // NS

BE = 3200


def _edge_feat_body(t_ref, src_ref, msg_ref, w_ref, wm_ref, bm_ref,
                    f_ref, srcm_ref):
    t = t_ref[...]
    mask = t <= T_MAX
    f = (jnp.cos(t * w_ref[...])
         + jnp.dot(msg_ref[...], wm_ref[...],
                   preferred_element_type=jnp.float32)
         + bm_ref[...])
    fb = jnp.where(mask, f, 0.0).astype(jnp.bfloat16)
    lo = jax.lax.bitcast_convert_type(fb[:, :64], jnp.int16)
    hi = jax.lax.bitcast_convert_type(fb[:, 64:], jnp.int16)
    f_ref[...] = (hi.astype(jnp.int32) << 16) | (lo.astype(jnp.int32)
                                                 & 0xFFFF)
    srcm_ref[...] = jnp.where(mask, src_ref[...], ZROW)


def _edge_features(t2, src2, msg, w_time, W_msg, bm):
    grid = N_EDGES // BE
    return pl.pallas_call(
        _edge_feat_body,
        grid=(grid,),
        in_specs=[
            pl.BlockSpec((BE, 1), lambda g: (g, 0)),
            pl.BlockSpec((BE, 1), lambda g: (g, 0)),
            pl.BlockSpec((BE, MSG_DIM), lambda g: (g, 0)),
            pl.BlockSpec((1, HIDDEN), lambda g: (0, 0)),
            pl.BlockSpec((MSG_DIM, HIDDEN), lambda g: (0, 0)),
            pl.BlockSpec((1, HIDDEN), lambda g: (0, 0)),
        ],
        out_specs=[
            pl.BlockSpec((BE, HIDDEN // 2), lambda g: (g, 0)),
            pl.BlockSpec((BE, 1), lambda g: (g, 0)),
        ],
        out_shape=[
            jax.ShapeDtypeStruct((N_EDGES, HIDDEN // 2), jnp.int32),
            jax.ShapeDtypeStruct((N_EDGES, 1), jnp.int32),
        ],
    )(t2, src2, msg, w_time, W_msg, bm)


def _bf_lo(x):
    return jax.lax.bitcast_convert_type(jax.lax.shift_left(x, 16),
                                        jnp.float32)


def _bf_hi(x):
    return jax.lax.bitcast_convert_type(x & jnp.int32(-65536), jnp.float32)


def _sc_agg_body(embsw_hbm, f_hbm, emb32_hbm, srcm_hbm, dst_hbm, idx_hbm,
                 zeros_hbm, inv_hbm, sel_out, acc, slotmap, srcv, dstv,
                 eidc, dstb, idxall, slotsel, rows0, fbuf0, rows1, fbuf1,
                 comb, selbuf, gsem0, fsem0, gsem1, fsem1):
    c = lax.axis_index("c")
    s = lax.axis_index("s")
    wid = c * NS + s
    pltpu.sync_copy(zeros_hbm.at[pl.ds(s * ROWS_PER_TILE, ROWS_PER_TILE), :],
                    acc.at[pl.ds(s * ROWS_PER_TILE, ROWS_PER_TILE), :])
    pltpu.sync_copy(inv_hbm, slotmap)
    pltpu.sync_copy(idx_hbm, idxall)
    lanes = jnp.arange(16, dtype=jnp.int32)

    def sbody(k, carry):
        vi = idxall[pl.ds(k * 16, 16)]
        plsc.store_scatter(slotmap, [vi], lanes + k * 16)
        return carry

    lax.fori_loop(0, BATCH // 16, sbody, 0)
    pltpu.sync_copy(srcm_hbm.at[wid], srcv.at[pl.ds(0, E_PER_W)])
    pltpu.sync_copy(dst_hbm.at[wid], dstv.at[pl.ds(0, E_PER_W)])

    base_e = wid * E_PER_W

    def cbody(i, off):
        d = dstv[pl.ds(i * 16, 16)]
        sv = srcv[pl.ds(i * 16, 16)]
        slot = plsc.load_gather(slotmap, [d])
        m = slot != TRASH
        plsc.store_compressed(dstv.at[pl.ds(off, 16)], slot, mask=m)
        plsc.store_compressed(srcv.at[pl.ds(off, 16)], sv, mask=m)
        plsc.store_compressed(eidc.at[pl.ds(off, 16)],
                              lanes + (base_e + i * 16), mask=m)
        return off + jnp.sum(m.astype(jnp.int32))

    cnt = lax.fori_loop(0, E_PER_W // 16, cbody, jnp.int32(0))

    def pbody(g, carry):
        dstv[pl.ds(cnt + g * 16, 16)] = jnp.full((16,), TRASH, jnp.int32)
        srcv[pl.ds(cnt + g * 16, 16)] = jnp.full((16,), ZROW, jnp.int32)
        eidc[pl.ds(cnt + g * 16, 16)] = jnp.zeros((16,), jnp.int32)
        return carry

    lax.fori_loop(0, 2 * CHUNK // 16, pbody, 0)
    plsc.subcore_barrier()

    def start_loads(k, rows, fbuf, gsem, fsem):
        pltpu.async_copy(embsw_hbm.at[srcv.at[pl.ds(k * CHUNK, CHUNK)]],
                         rows, gsem)
        pltpu.async_copy(f_hbm.at[eidc.at[pl.ds(k * CHUNK, CHUNK)]],
                         fbuf, fsem)

    def wait_loads(k, rows, fbuf, gsem, fsem):
        pltpu.make_async_copy(embsw_hbm.at[srcv.at[pl.ds(k * CHUNK, CHUNK)]],
                              rows, gsem).wait()
        pltpu.make_async_copy(f_hbm.at[eidc.at[pl.ds(k * CHUNK, CHUNK)]],
                              fbuf, fsem).wait()

    def scatter(k, rows, fbuf):
        for v in range(CHUNK // 16):
            dstb[pl.ds(v * 16, 16)] = dstv[pl.ds(k * CHUNK + v * 16, 16)]

        def gbody(i, carry):
            r = i // (HIDDEN // 32)
            g = i % (HIDDEN // 32)
            xr = rows[r, pl.ds(g * 16, 16)]
            xf = fbuf[r, pl.ds(g * 16, 16)]
            comb[r, pl.ds(g * 16, 16)] = _bf_lo(xr) + _bf_lo(xf)
            comb[r, pl.ds(64 + g * 16, 16)] = _bf_hi(xr) + _bf_hi(xf)
            return carry

        lax.fori_loop(0, CHUNK * (HIDDEN // 32), gbody, 0)
        pltpu.sync_copy(comb, acc.at[dstb], add=True)

    npairs = (cnt + CHUNK - 1) // CHUNK // 2

    start_loads(0, rows0, fbuf0, gsem0, fsem0)

    def body(j, carry):
        a = 2 * j
        start_loads(a + 1, rows1, fbuf1, gsem1, fsem1)
        wait_loads(a, rows0, fbuf0, gsem0, fsem0)
        scatter(a, rows0, fbuf0)
        start_loads(a + 2, rows0, fbuf0, gsem0, fsem0)
        wait_loads(a + 1, rows1, fbuf1, gsem1, fsem1)
        scatter(a + 1, rows1, fbuf1)
        return carry

    lax.fori_loop(0, npairs, body, 0)
    wait_loads(2 * npairs, rows0, fbuf0, gsem0, fsem0)
    scatter(2 * npairs, rows0, fbuf0)
    plsc.subcore_barrier()

    for k in range(B_PER_TILE // 16):
        vi = idxall[pl.ds(s * B_PER_TILE + k * 16, 16)]
        slotsel[pl.ds(k * 16, 16)] = plsc.load_gather(slotmap, [vi])
    pltpu.sync_copy(acc.at[slotsel], selbuf)
    pltpu.sync_copy(selbuf, sel_out.at[c, pl.ds(s * B_PER_TILE, B_PER_TILE), :])

    @pl.when(c == 0)
    def _():
        pltpu.async_copy(
            emb32_hbm.at[idxall.at[pl.ds(s * B_PER_TILE, B_PER_TILE)]],
            selbuf, gsem0).wait()
        pltpu.sync_copy(selbuf,
                        sel_out.at[2, pl.ds(s * B_PER_TILE, B_PER_TILE), :])


@functools.cache
def _make_sc_agg():
    return functools.partial(
        pl.kernel,
        out_type=jax.ShapeDtypeStruct((3, BATCH, HIDDEN), jnp.float32),
        mesh=plsc.VectorSubcoreMesh(core_axis_name="c", subcore_axis_name="s",
                                    num_cores=NC, num_subcores=NS),
        compiler_params=pltpu.CompilerParams(use_tc_tiling_on_sc=False,
                                             needs_layout_passes=False),
        scratch_types=[
            pltpu.VMEM_SHARED((N_ACC, HIDDEN), jnp.float32),
            pltpu.VMEM((N_NODES,), jnp.int32),
            pltpu.VMEM((E_PER_W + 2 * CHUNK,), jnp.int32),
            pltpu.VMEM((E_PER_W + 2 * CHUNK,), jnp.int32),
            pltpu.VMEM((E_PER_W + 2 * CHUNK,), jnp.int32),
            pltpu.VMEM((CHUNK,), jnp.int32),
            pltpu.VMEM((BATCH,), jnp.int32),
            pltpu.VMEM((B_PER_TILE,), jnp.int32),
            pltpu.VMEM((CHUNK, HIDDEN // 2), jnp.int32),
            pltpu.VMEM((CHUNK, HIDDEN // 2), jnp.int32),
            pltpu.VMEM((CHUNK, HIDDEN // 2), jnp.int32),
            pltpu.VMEM((CHUNK, HIDDEN // 2), jnp.int32),
            pltpu.VMEM((CHUNK, HIDDEN), jnp.float32),
            pltpu.VMEM((B_PER_TILE, HIDDEN), jnp.float32),
            pltpu.SemaphoreType.DMA,
            pltpu.SemaphoreType.DMA,
            pltpu.SemaphoreType.DMA,
            pltpu.SemaphoreType.DMA,
        ],
    )(_sc_agg_body)


def _cls_body(sel_ref, wu_ref, bu_ref, w1_ref, b1_ref, w2_ref, b2_ref,
              out_ref):
    x = sel_ref[0] + sel_ref[1] + sel_ref[2]
    h = jnp.maximum(
        jnp.dot(x, wu_ref[...], preferred_element_type=jnp.float32)
        + bu_ref[...], 0.0)
    h2 = jnp.maximum(
        jnp.dot(h, w1_ref[...], preferred_element_type=jnp.float32)
        + b1_ref[...], 0.0)
    out_ref[...] = (jnp.dot(h2, w2_ref[...],
                            preferred_element_type=jnp.float32)
                    + b2_ref[...])


def _classifier(sel, W_upd, bu, W1, b1, W2, b2):
    return pl.pallas_call(
        _cls_body,
        out_shape=jax.ShapeDtypeStruct((BATCH, HIDDEN), jnp.float32),
    )(sel, W_upd, bu, W1, b1, W2, b2)


def kernel(src, dst, t, msg, labels, idx, node_emb, w_time, W_msg, b_msg,
           W_upd, b_upd, W1, b1, W2, b2):
    del labels
    t2 = t.reshape(N_EDGES, 1)
    src2 = src.reshape(N_EDGES, 1)
    F, srcm = _edge_features(t2, src2, msg, w_time, W_msg,
                             b_msg.reshape(1, HIDDEN))
    emb_pad = jnp.concatenate(
        [node_emb, jnp.zeros((N_PAD - N_NODES, HIDDEN), node_emb.dtype)],
        axis=0)
    eb = node_emb.astype(jnp.bfloat16)
    lo = (jax.lax.bitcast_convert_type(eb[:, :64], jnp.int16)
          .astype(jnp.int32) & 0xFFFF)
    hi = jax.lax.bitcast_convert_type(eb[:, 64:], jnp.int16).astype(
        jnp.int32) << 16
    emb_sw = jnp.concatenate(
        [hi | lo, jnp.zeros((N_PAD - N_NODES, HIDDEN // 2), jnp.int32)],
        axis=0)
    sel = _make_sc_agg()(emb_sw, F, emb_pad,
                  srcm.reshape(NW, E_PER_W),
                  dst.reshape(NW, E_PER_W),
                  idx,
                  jnp.zeros((N_ACC, HIDDEN), jnp.float32),
                  jnp.full((N_NODES,), TRASH, jnp.int32))
    return _classifier(sel, W_upd, b_upd.reshape(1, HIDDEN), W1,
                       b1.reshape(1, HIDDEN), W2, b2.reshape(1, HIDDEN))


# speedup vs baseline: 5.6985x; 1.8413x over previous
import functools

import jax
import jax.numpy as jnp
from jax import lax
from jax.experimental import pallas as pl
from jax.experimental.pallas import tpu as pltpu
from jax.experimental.pallas import tpu_sc as plsc

N_NODES = 10000
N_EDGES = 320000
HIDDEN = 128
MSG_DIM = 16
BATCH = 2048
T_MAX = 1000.0

NC, NS = 2, 16
NW = NC * NS
E_PER_W = N_EDGES // NW
CHUNK = 80
PADC = 2 * CHUNK
CAP = E_PER_W + PADC
B_PER_TILE = BATCH // NS
TRASH = BATCH
N_ACC = 2176
ROWS_PER_TILE = N_ACC // NS

BE = 3200
REG = 12800
KBLK = REG // BE

_SC_PARAMS = pltpu.CompilerParams(use_tc_tiling_on_sc=False,
                                  needs_layout_passes=False)


def _sc_compact_body(src_hbm, dst_hbm, t_hbm, idx_hbm, inv_hbm,
                     srcc_hbm, slotc_hbm, eidc_hbm, tcc_hbm, cnts_hbm,
                     slotsel_hbm,
                     slotmap, srcv, dstv, tvv, eidv, idxall, cntbuf, sselbuf):
    c = lax.axis_index("c")
    s = lax.axis_index("s")
    wid = c * NS + s
    pltpu.sync_copy(inv_hbm, slotmap)
    pltpu.sync_copy(idx_hbm, idxall)
    lanes = jnp.arange(16, dtype=jnp.int32)

    def sbody(k, carry):
        vi = idxall[pl.ds(k * 16, 16)]
        plsc.store_scatter(slotmap, [vi], lanes + k * 16)
        return carry

    lax.fori_loop(0, BATCH // 16, sbody, 0)
    pltpu.sync_copy(src_hbm.at[wid], srcv.at[pl.ds(0, E_PER_W)])
    pltpu.sync_copy(dst_hbm.at[wid], dstv.at[pl.ds(0, E_PER_W)])
    pltpu.sync_copy(t_hbm.at[wid], tvv.at[pl.ds(0, E_PER_W)])

    base_e = wid * E_PER_W

    def cbody(i, off):
        d = dstv[pl.ds(i * 16, 16)]
        sv = srcv[pl.ds(i * 16, 16)]
        tval = tvv[pl.ds(i * 16, 16)]
        slot = plsc.load_gather(slotmap, [d])
        m = (slot != TRASH) & (tval <= T_MAX)
        plsc.store_compressed(dstv.at[pl.ds(off, 16)], slot, mask=m)
        plsc.store_compressed(srcv.at[pl.ds(off, 16)], sv, mask=m)
        plsc.store_compressed(tvv.at[pl.ds(off, 16)], tval, mask=m)
        plsc.store_compressed(eidv.at[pl.ds(off, 16)],
                              lanes + (base_e + i * 16), mask=m)
        return off + jnp.sum(m.astype(jnp.int32))

    cnt = lax.fori_loop(0, E_PER_W // 16, cbody, jnp.int32(0))

    def pbody(g, carry):
        dstv[pl.ds(cnt + g * 16, 16)] = jnp.full((16,), TRASH, jnp.int32)
        srcv[pl.ds(cnt + g * 16, 16)] = jnp.zeros((16,), jnp.int32)
        tvv[pl.ds(cnt + g * 16, 16)] = jnp.zeros((16,), jnp.float32)
        eidv[pl.ds(cnt + g * 16, 16)] = jnp.zeros((16,), jnp.int32)
        return carry

    lax.fori_loop(0, PADC // 16, pbody, 0)

    pltpu.sync_copy(srcv, srcc_hbm.at[wid, pl.ds(0, CAP)])
    pltpu.sync_copy(dstv, slotc_hbm.at[wid, pl.ds(0, CAP)])
    pltpu.sync_copy(eidv, eidc_hbm.at[wid, pl.ds(0, CAP)])
    pltpu.sync_copy(tvv, tcc_hbm.at[wid, pl.ds(0, CAP)])
    cntbuf[pl.ds(0, 16)] = jnp.zeros((16,), jnp.int32) + cnt
    pltpu.sync_copy(cntbuf, cnts_hbm.at[wid])

    for k in range(B_PER_TILE // 16):
        vi = idxall[pl.ds(s * B_PER_TILE + k * 16, 16)]
        sselbuf[pl.ds(k * 16, 16)] = plsc.load_gather(slotmap, [vi])

    @pl.when(c == 0)
    def _():
        pltpu.sync_copy(sselbuf,
                        slotsel_hbm.at[pl.ds(s * B_PER_TILE, B_PER_TILE)])


@functools.cache
def _make_sc_compact():
    return functools.partial(
        pl.kernel,
        out_type=(
            jax.ShapeDtypeStruct((NW, REG), jnp.int32),
            jax.ShapeDtypeStruct((NW, REG), jnp.int32),
            jax.ShapeDtypeStruct((NW, REG), jnp.int32),
            jax.ShapeDtypeStruct((NW, REG), jnp.float32),
            jax.ShapeDtypeStruct((NW, 16), jnp.int32),
            jax.ShapeDtypeStruct((BATCH,), jnp.int32),
        ),
        mesh=plsc.VectorSubcoreMesh(core_axis_name="c", subcore_axis_name="s",
                                    num_cores=NC, num_subcores=NS),
        compiler_params=_SC_PARAMS,
        scratch_types=[
            pltpu.VMEM((N_NODES,), jnp.int32),
            pltpu.VMEM((CAP,), jnp.int32),
            pltpu.VMEM((CAP,), jnp.int32),
            pltpu.VMEM((CAP,), jnp.float32),
            pltpu.VMEM((CAP,), jnp.int32),
            pltpu.VMEM((BATCH,), jnp.int32),
            pltpu.VMEM((16,), jnp.int32),
            pltpu.VMEM((B_PER_TILE,), jnp.int32),
        ],
    )(_sc_compact_body)


def _cos_body(cnts_ref, t_ref, w_ref, bm_ref, f_ref):
    g = pl.program_id(0)
    w = g // KBLK

    @pl.when((g % KBLK) * BE < cnts_ref[w])
    def _():
        f = jnp.cos(t_ref[...] * w_ref[...]) + bm_ref[...]
        fb = f.astype(jnp.bfloat16)
        lo = jax.lax.bitcast_convert_type(fb[:, :64], jnp.int16)
        hi = jax.lax.bitcast_convert_type(fb[:, 64:], jnp.int16)
        f_ref[...] = (hi.astype(jnp.int32) << 16) | (lo.astype(jnp.int32)
                                                     & 0xFFFF)


def _cos_features(cnts32, t_c, w_time, bm):
    return pl.pallas_call(
        _cos_body,
        grid_spec=pltpu.PrefetchScalarGridSpec(
            num_scalar_prefetch=1,
            grid=(NW * KBLK,),
            in_specs=[
                pl.BlockSpec((BE, 1), lambda g, cnts: (g, 0)),
                pl.BlockSpec((1, HIDDEN), lambda g, cnts: (0, 0)),
                pl.BlockSpec((1, HIDDEN), lambda g, cnts: (0, 0)),
            ],
            out_specs=pl.BlockSpec((BE, HIDDEN // 2), lambda g, cnts: (g, 0)),
        ),
        out_shape=jax.ShapeDtypeStruct((NW * REG, HIDDEN // 2), jnp.int32),
    )(cnts32, t_c, w_time, bm)


def _bf_lo(x):
    return jax.lax.bitcast_convert_type(jax.lax.shift_left(x, 16),
                                        jnp.float32)


def _bf_hi(x):
    return jax.lax.bitcast_convert_type(x & jnp.int32(-65536), jnp.float32)


def _sc_agg_body(embsw_hbm, f_hbm, msg_hbm, emb32_hbm, srcc_hbm, slotc_hbm,
                 eidc_hbm, cnts_hbm, slotsel_hbm, idx_hbm, zeros_hbm,
                 mzeros_hbm, sel_out, msel_out,
                 acc, macc, srcv, dstv, eidv, dstb, cntbuf, sselbuf, ibuf,
                 rows0, fbuf0, mrow0, rows1, fbuf1, mrow1, comb, selbuf,
                 mselbuf, gsem0, fsem0, msem0, gsem1, fsem1, msem1):
    c = lax.axis_index("c")
    s = lax.axis_index("s")
    wid = c * NS + s
    lanes = jnp.arange(16, dtype=jnp.int32)
    pltpu.sync_copy(zeros_hbm.at[pl.ds(s * ROWS_PER_TILE, ROWS_PER_TILE), :],
                    acc.at[pl.ds(s * ROWS_PER_TILE, ROWS_PER_TILE), :])
    pltpu.sync_copy(mzeros_hbm.at[pl.ds(s * ROWS_PER_TILE, ROWS_PER_TILE), :],
                    macc.at[pl.ds(s * ROWS_PER_TILE, ROWS_PER_TILE), :])
    pltpu.sync_copy(srcc_hbm.at[wid, pl.ds(0, CAP)], srcv)
    pltpu.sync_copy(slotc_hbm.at[wid, pl.ds(0, CAP)], dstv)
    pltpu.sync_copy(eidc_hbm.at[wid, pl.ds(0, CAP)], eidv)
    pltpu.sync_copy(cnts_hbm.at[wid], cntbuf)
    cnt = jnp.sum(jnp.where(lanes == 0, cntbuf[pl.ds(0, 16)], 0))
    plsc.subcore_barrier()

    fbase = wid * REG

    def start_loads(k, rows, fbuf, mrow, gsem, fsem, msem):
        pltpu.async_copy(embsw_hbm.at[srcv.at[pl.ds(k * CHUNK, CHUNK)]],
                         rows, gsem)
        pltpu.async_copy(f_hbm.at[pl.ds(fbase + k * CHUNK, CHUNK), :],
                         fbuf, fsem)
        pltpu.async_copy(msg_hbm.at[eidv.at[pl.ds(k * CHUNK, CHUNK)]],
                         mrow, msem)

    def wait_loads(k, rows, fbuf, mrow, gsem, fsem, msem):
        pltpu.make_async_copy(embsw_hbm.at[srcv.at[pl.ds(k * CHUNK, CHUNK)]],
                              rows, gsem).wait()
        pltpu.make_async_copy(f_hbm.at[pl.ds(fbase + k * CHUNK, CHUNK), :],
                              fbuf, fsem).wait()
        pltpu.make_async_copy(msg_hbm.at[eidv.at[pl.ds(k * CHUNK, CHUNK)]],
                              mrow, msem).wait()

    def scatter(k, rows, fbuf, mrow):
        for v in range(CHUNK // 16):
            dstb[pl.ds(v * 16, 16)] = dstv[pl.ds(k * CHUNK + v * 16, 16)]

        def gbody(i, carry):
            r = i // (HIDDEN // 32)
            gq = i % (HIDDEN // 32)
            xr = rows[r, pl.ds(gq * 16, 16)]
            xf = fbuf[r, pl.ds(gq * 16, 16)]
            comb[r, pl.ds(gq * 16, 16)] = _bf_lo(xr) + _bf_lo(xf)
            comb[r, pl.ds(64 + gq * 16, 16)] = _bf_hi(xr) + _bf_hi(xf)
            return carry

        lax.fori_loop(0, CHUNK * (HIDDEN // 32), gbody, 0)
        pltpu.sync_copy(comb, acc.at[dstb], add=True)
        pltpu.sync_copy(mrow, macc.at[dstb], add=True)

    npairs = (cnt + CHUNK - 1) // CHUNK // 2

    start_loads(0, rows0, fbuf0, mrow0, gsem0, fsem0, msem0)

    def body(j, carry):
        a = 2 * j
        start_loads(a + 1, rows1, fbuf1, mrow1, gsem1, fsem1, msem1)
        wait_loads(a, rows0, fbuf0, mrow0, gsem0, fsem0, msem0)
        scatter(a, rows0, fbuf0, mrow0)
        start_loads(a + 2, rows0, fbuf0, mrow0, gsem0, fsem0, msem0)
        wait_loads(a + 1, rows1, fbuf1, mrow1, gsem1, fsem1, msem1)
        scatter(a + 1, rows1, fbuf1, mrow1)
        return carry

    lax.fori_loop(0, npairs, body, 0)
    wait_loads(2 * npairs, rows0, fbuf0, mrow0, gsem0, fsem0, msem0)
    scatter(2 * npairs, rows0, fbuf0, mrow0)
    plsc.subcore_barrier()

    pltpu.sync_copy(slotsel_hbm.at[pl.ds(s * B_PER_TILE, B_PER_TILE)],
                    sselbuf)
    pltpu.sync_copy(acc.at[sselbuf], selbuf)
    pltpu.sync_copy(selbuf, sel_out.at[c, pl.ds(s * B_PER_TILE, B_PER_TILE), :])
    pltpu.sync_copy(macc.at[sselbuf], mselbuf)
    pltpu.sync_copy(mselbuf,
                    msel_out.at[c, pl.ds(s * B_PER_TILE, B_PER_TILE), :])

    @pl.when(c == 0)
    def _():
        pltpu.sync_copy(idx_hbm.at[pl.ds(s * B_PER_TILE, B_PER_TILE)], ibuf)
        pltpu.async_copy(emb32_hbm.at[ibuf], selbuf, gsem0).wait()
        pltpu.sync_copy(selbuf,
                        sel_out.at[2, pl.ds(s * B_PER_TILE, B_PER_TILE), :])


@functools.cache
def _make_sc_agg():
    return functools.partial(
        pl.kernel,
        out_type=(
            jax.ShapeDtypeStruct((3, BATCH, HIDDEN), jnp.float32),
            jax.ShapeDtypeStruct((2, BATCH, MSG_DIM), jnp.float32),
        ),
        mesh=plsc.VectorSubcoreMesh(core_axis_name="c", subcore_axis_name="s",
                                    num_cores=NC, num_subcores=NS),
        compiler_params=_SC_PARAMS,
        scratch_types=[
            pltpu.VMEM_SHARED((N_ACC, HIDDEN), jnp.float32),
            pltpu.VMEM_SHARED((N_ACC, MSG_DIM), jnp.float32),
            pltpu.VMEM((CAP,), jnp.int32),
            pltpu.VMEM((CAP,), jnp.int32),
            pltpu.VMEM((CAP,), jnp.int32),
            pltpu.VMEM((CHUNK,), jnp.int32),
            pltpu.VMEM((16,), jnp.int32),
            pltpu.VMEM((B_PER_TILE,), jnp.int32),
            pltpu.VMEM((B_PER_TILE,), jnp.int32),
            pltpu.VMEM((CHUNK, HIDDEN // 2), jnp.int32),
            pltpu.VMEM((CHUNK, HIDDEN // 2), jnp.int32),
            pltpu.VMEM((CHUNK, MSG_DIM), jnp.float32),
            pltpu.VMEM((CHUNK, HIDDEN // 2), jnp.int32),
            pltpu.VMEM((CHUNK, HIDDEN // 2), jnp.int32),
            pltpu.VMEM((CHUNK, MSG_DIM), jnp.float32),
            pltpu.VMEM((CHUNK, HIDDEN), jnp.float32),
            pltpu.VMEM((B_PER_TILE, HIDDEN), jnp.float32),
            pltpu.VMEM((B_PER_TILE, MSG_DIM), jnp.float32),
            pltpu.SemaphoreType.DMA,
            pltpu.SemaphoreType.DMA,
            pltpu.SemaphoreType.DMA,
            pltpu.SemaphoreType.DMA,
            pltpu.SemaphoreType.DMA,
            pltpu.SemaphoreType.DMA,
        ],
    )(_sc_agg_body)


def _cls_body(sel_ref, msel_ref, wm_ref, wu_ref, bu_ref, w1_ref, b1_ref,
              w2_ref, b2_ref, out_ref):
    x = (sel_ref[0] + sel_ref[1] + sel_ref[2]
         + jnp.dot(msel_ref[0] + msel_ref[1], wm_ref[...],
                   preferred_element_type=jnp.float32))
    h = jnp.maximum(
        jnp.dot(x, wu_ref[...], preferred_element_type=jnp.float32)
        + bu_ref[...], 0.0)
    h2 = jnp.maximum(
        jnp.dot(h, w1_ref[...], preferred_element_type=jnp.float32)
        + b1_ref[...], 0.0)
    out_ref[...] = (jnp.dot(h2, w2_ref[...],
                            preferred_element_type=jnp.float32)
                    + b2_ref[...])


def _classifier(sel, msel, W_msg, W_upd, bu, W1, b1, W2, b2):
    return pl.pallas_call(
        _cls_body,
        out_shape=jax.ShapeDtypeStruct((BATCH, HIDDEN), jnp.float32),
    )(sel, msel, W_msg, W_upd, bu, W1, b1, W2, b2)


def kernel(src, dst, t, msg, labels, idx, node_emb, w_time, W_msg, b_msg,
           W_upd, b_upd, W1, b1, W2, b2):
    del labels
    srcc, slotc, eidc, tcc, cnts, slotsel = _make_sc_compact()(
        src.reshape(NW, E_PER_W),
        dst.reshape(NW, E_PER_W),
        t.reshape(NW, E_PER_W),
        idx,
        jnp.full((N_NODES,), TRASH, jnp.int32))
    F = _cos_features(cnts[:, 0], tcc.reshape(NW * REG, 1), w_time,
                      b_msg.reshape(1, HIDDEN))
    eb = node_emb.astype(jnp.bfloat16)
    lo = (jax.lax.bitcast_convert_type(eb[:, :64], jnp.int16)
          .astype(jnp.int32) & 0xFFFF)
    hi = jax.lax.bitcast_convert_type(eb[:, 64:], jnp.int16).astype(
        jnp.int32) << 16
    emb_sw = hi | lo
    sel, msel = _make_sc_agg()(
        emb_sw, F, msg, node_emb, srcc, slotc, eidc, cnts, slotsel, idx,
        jnp.zeros((N_ACC, HIDDEN), jnp.float32),
        jnp.zeros((N_ACC, MSG_DIM), jnp.float32))
    return _classifier(sel, msel, W_msg, W_upd, b_upd.reshape(1, HIDDEN),
                       W1, b1.reshape(1, HIDDEN), W2, b2.reshape(1, HIDDEN))


# speedup vs baseline: 5.7922x; 1.0164x over previous
import functools

import jax
import jax.numpy as jnp
from jax import lax
from jax.experimental import pallas as pl
from jax.experimental.pallas import tpu as pltpu
from jax.experimental.pallas import tpu_sc as plsc

N_NODES = 10000
N_EDGES = 320000
HIDDEN = 128
MSG_DIM = 16
BATCH = 2048
T_MAX = 1000.0

NC, NS = 2, 16
NW = NC * NS
E_PER_W = N_EDGES // NW
CHUNK = 80
PADC = 2 * CHUNK
CAP = E_PER_W + PADC
B_PER_TILE = BATCH // NS
TRASH = BATCH
N_ACC = 2176
ROWS_PER_TILE = N_ACC // NS

BE = 640
REG = 10240
KBLK = REG // BE

_SC_PARAMS = pltpu.CompilerParams(use_tc_tiling_on_sc=False,
                                  needs_layout_passes=False)


def _sc_compact_body(src_hbm, dst_hbm, t_hbm, idx_hbm, inv_hbm,
                     srcc_hbm, slotc_hbm, eidc_hbm, tcc_hbm, cnts_hbm,
                     slotsel_hbm,
                     slotmap, srcv, dstv, tvv, eidv, idxall, cntbuf, sselbuf):
    c = lax.axis_index("c")
    s = lax.axis_index("s")
    wid = c * NS + s
    pltpu.sync_copy(inv_hbm, slotmap)
    pltpu.sync_copy(idx_hbm, idxall)
    lanes = jnp.arange(16, dtype=jnp.int32)

    def sbody(k, carry):
        vi = idxall[pl.ds(k * 16, 16)]
        plsc.store_scatter(slotmap, [vi], lanes + k * 16)
        return carry

    lax.fori_loop(0, BATCH // 16, sbody, 0)
    pltpu.sync_copy(src_hbm.at[wid], srcv.at[pl.ds(0, E_PER_W)])
    pltpu.sync_copy(dst_hbm.at[wid], dstv.at[pl.ds(0, E_PER_W)])
    pltpu.sync_copy(t_hbm.at[wid], tvv.at[pl.ds(0, E_PER_W)])

    base_e = wid * E_PER_W

    def cbody(i, off):
        d = dstv[pl.ds(i * 16, 16)]
        sv = srcv[pl.ds(i * 16, 16)]
        tval = tvv[pl.ds(i * 16, 16)]
        slot = plsc.load_gather(slotmap, [d])
        m = (slot != TRASH) & (tval <= T_MAX)
        plsc.store_compressed(dstv.at[pl.ds(off, 16)], slot, mask=m)
        plsc.store_compressed(srcv.at[pl.ds(off, 16)], sv, mask=m)
        plsc.store_compressed(tvv.at[pl.ds(off, 16)], tval, mask=m)
        plsc.store_compressed(eidv.at[pl.ds(off, 16)],
                              lanes + (base_e + i * 16), mask=m)
        return off + jnp.sum(m.astype(jnp.int32))

    cnt = lax.fori_loop(0, E_PER_W // 16, cbody, jnp.int32(0))

    def pbody(g, carry):
        dstv[pl.ds(cnt + g * 16, 16)] = jnp.full((16,), TRASH, jnp.int32)
        srcv[pl.ds(cnt + g * 16, 16)] = jnp.zeros((16,), jnp.int32)
        tvv[pl.ds(cnt + g * 16, 16)] = jnp.zeros((16,), jnp.float32)
        eidv[pl.ds(cnt + g * 16, 16)] = jnp.zeros((16,), jnp.int32)
        return carry

    lax.fori_loop(0, PADC // 16, pbody, 0)

    pltpu.sync_copy(srcv, srcc_hbm.at[wid, pl.ds(0, CAP)])
    pltpu.sync_copy(dstv, slotc_hbm.at[wid, pl.ds(0, CAP)])
    pltpu.sync_copy(eidv, eidc_hbm.at[wid, pl.ds(0, CAP)])
    pltpu.sync_copy(tvv, tcc_hbm.at[wid, pl.ds(0, CAP)])
    cntbuf[pl.ds(0, 16)] = jnp.zeros((16,), jnp.int32) + cnt
    pltpu.sync_copy(cntbuf, cnts_hbm.at[wid])

    for k in range(B_PER_TILE // 16):
        vi = idxall[pl.ds(s * B_PER_TILE + k * 16, 16)]
        sselbuf[pl.ds(k * 16, 16)] = plsc.load_gather(slotmap, [vi])

    @pl.when(c == 0)
    def _():
        pltpu.sync_copy(sselbuf,
                        slotsel_hbm.at[pl.ds(s * B_PER_TILE, B_PER_TILE)])


@functools.cache
def _make_sc_compact():
    return functools.partial(
        pl.kernel,
        out_type=(
            jax.ShapeDtypeStruct((NW, REG), jnp.int32),
            jax.ShapeDtypeStruct((NW, REG), jnp.int32),
            jax.ShapeDtypeStruct((NW, REG), jnp.int32),
            jax.ShapeDtypeStruct((NW, REG), jnp.float32),
            jax.ShapeDtypeStruct((NW, 16), jnp.int32),
            jax.ShapeDtypeStruct((BATCH,), jnp.int32),
        ),
        mesh=plsc.VectorSubcoreMesh(core_axis_name="c", subcore_axis_name="s",
                                    num_cores=NC, num_subcores=NS),
        compiler_params=_SC_PARAMS,
        scratch_types=[
            pltpu.VMEM((N_NODES,), jnp.int32),
            pltpu.VMEM((CAP,), jnp.int32),
            pltpu.VMEM((CAP,), jnp.int32),
            pltpu.VMEM((CAP,), jnp.float32),
            pltpu.VMEM((CAP,), jnp.int32),
            pltpu.VMEM((BATCH,), jnp.int32),
            pltpu.VMEM((16,), jnp.int32),
            pltpu.VMEM((B_PER_TILE,), jnp.int32),
        ],
    )(_sc_compact_body)


def _cos_body(cnts_ref, t_ref, w_ref, bm_ref, f_ref):
    g = pl.program_id(0)
    w = g // KBLK

    @pl.when((g % KBLK) * BE < cnts_ref[w])
    def _():
        f = jnp.cos(t_ref[...] * w_ref[...]) + bm_ref[...]
        fb = f.astype(jnp.bfloat16)
        lo = jax.lax.bitcast_convert_type(fb[:, :64], jnp.int16)
        hi = jax.lax.bitcast_convert_type(fb[:, 64:], jnp.int16)
        f_ref[...] = (hi.astype(jnp.int32) << 16) | (lo.astype(jnp.int32)
                                                     & 0xFFFF)


def _cos_features(cnts32, t_c, w_time, bm):
    return pl.pallas_call(
        _cos_body,
        grid_spec=pltpu.PrefetchScalarGridSpec(
            num_scalar_prefetch=1,
            grid=(NW * KBLK,),
            in_specs=[
                pl.BlockSpec((BE, 1), lambda g, cnts: (g, 0)),
                pl.BlockSpec((1, HIDDEN), lambda g, cnts: (0, 0)),
                pl.BlockSpec((1, HIDDEN), lambda g, cnts: (0, 0)),
            ],
            out_specs=pl.BlockSpec((BE, HIDDEN // 2), lambda g, cnts: (g, 0)),
        ),
        out_shape=jax.ShapeDtypeStruct((NW * REG, HIDDEN // 2), jnp.int32),
    )(cnts32, t_c, w_time, bm)


def _bf_lo(x):
    return jax.lax.bitcast_convert_type(jax.lax.shift_left(x, 16),
                                        jnp.float32)


def _bf_hi(x):
    return jax.lax.bitcast_convert_type(x & jnp.int32(-65536), jnp.float32)


def _sc_agg_body(embsw_hbm, f_hbm, msg_hbm, emb32_hbm, srcc_hbm, slotc_hbm,
                 eidc_hbm, cnts_hbm, slotsel_hbm, idx_hbm, zeros_hbm,
                 mzeros_hbm, sel_out, msel_out,
                 acc, macc, srcv, dstv, eidv, dstb, cntbuf, sselbuf, ibuf,
                 rows0, fbuf0, mrow0, rows1, fbuf1, mrow1, comb, selbuf,
                 mselbuf, gsem0, fsem0, msem0, gsem1, fsem1, msem1):
    c = lax.axis_index("c")
    s = lax.axis_index("s")
    wid = c * NS + s
    lanes = jnp.arange(16, dtype=jnp.int32)
    pltpu.sync_copy(zeros_hbm.at[pl.ds(s * ROWS_PER_TILE, ROWS_PER_TILE), :],
                    acc.at[pl.ds(s * ROWS_PER_TILE, ROWS_PER_TILE), :])
    pltpu.sync_copy(mzeros_hbm.at[pl.ds(s * ROWS_PER_TILE, ROWS_PER_TILE), :],
                    macc.at[pl.ds(s * ROWS_PER_TILE, ROWS_PER_TILE), :])
    pltpu.sync_copy(srcc_hbm.at[wid, pl.ds(0, CAP)], srcv)
    pltpu.sync_copy(slotc_hbm.at[wid, pl.ds(0, CAP)], dstv)
    pltpu.sync_copy(eidc_hbm.at[wid, pl.ds(0, CAP)], eidv)
    pltpu.sync_copy(cnts_hbm.at[wid], cntbuf)
    cnt = jnp.sum(jnp.where(lanes == 0, cntbuf[pl.ds(0, 16)], 0))
    plsc.subcore_barrier()

    fbase = wid * REG

    def start_loads(k, rows, fbuf, mrow, gsem, fsem, msem):
        pltpu.async_copy(embsw_hbm.at[srcv.at[pl.ds(k * CHUNK, CHUNK)]],
                         rows, gsem)
        pltpu.async_copy(f_hbm.at[pl.ds(fbase + k * CHUNK, CHUNK), :],
                         fbuf, fsem)
        pltpu.async_copy(msg_hbm.at[eidv.at[pl.ds(k * CHUNK, CHUNK)]],
                         mrow, msem)

    def wait_loads(k, rows, fbuf, mrow, gsem, fsem, msem):
        pltpu.make_async_copy(embsw_hbm.at[srcv.at[pl.ds(k * CHUNK, CHUNK)]],
                              rows, gsem).wait()
        pltpu.make_async_copy(f_hbm.at[pl.ds(fbase + k * CHUNK, CHUNK), :],
                              fbuf, fsem).wait()
        pltpu.make_async_copy(msg_hbm.at[eidv.at[pl.ds(k * CHUNK, CHUNK)]],
                              mrow, msem).wait()

    def scatter(k, rows, fbuf, mrow):
        for v in range(CHUNK // 16):
            dstb[pl.ds(v * 16, 16)] = dstv[pl.ds(k * CHUNK + v * 16, 16)]

        def gbody(i, carry):
            r = i // (HIDDEN // 32)
            gq = i % (HIDDEN // 32)
            xr = rows[r, pl.ds(gq * 16, 16)]
            xf = fbuf[r, pl.ds(gq * 16, 16)]
            comb[r, pl.ds(gq * 16, 16)] = _bf_lo(xr) + _bf_lo(xf)
            comb[r, pl.ds(64 + gq * 16, 16)] = _bf_hi(xr) + _bf_hi(xf)
            return carry

        lax.fori_loop(0, CHUNK * (HIDDEN // 32), gbody, 0)
        pltpu.sync_copy(comb, acc.at[dstb], add=True)
        pltpu.sync_copy(mrow, macc.at[dstb], add=True)

    npairs = (cnt + CHUNK - 1) // CHUNK // 2

    start_loads(0, rows0, fbuf0, mrow0, gsem0, fsem0, msem0)

    def body(j, carry):
        a = 2 * j
        start_loads(a + 1, rows1, fbuf1, mrow1, gsem1, fsem1, msem1)
        wait_loads(a, rows0, fbuf0, mrow0, gsem0, fsem0, msem0)
        scatter(a, rows0, fbuf0, mrow0)
        start_loads(a + 2, rows0, fbuf0, mrow0, gsem0, fsem0, msem0)
        wait_loads(a + 1, rows1, fbuf1, mrow1, gsem1, fsem1, msem1)
        scatter(a + 1, rows1, fbuf1, mrow1)
        return carry

    lax.fori_loop(0, npairs, body, 0)
    wait_loads(2 * npairs, rows0, fbuf0, mrow0, gsem0, fsem0, msem0)
    scatter(2 * npairs, rows0, fbuf0, mrow0)
    plsc.subcore_barrier()

    pltpu.sync_copy(slotsel_hbm.at[pl.ds(s * B_PER_TILE, B_PER_TILE)],
                    sselbuf)
    pltpu.sync_copy(acc.at[sselbuf], selbuf)
    pltpu.sync_copy(selbuf, sel_out.at[c, pl.ds(s * B_PER_TILE, B_PER_TILE), :])
    pltpu.sync_copy(macc.at[sselbuf], mselbuf)
    pltpu.sync_copy(mselbuf,
                    msel_out.at[c, pl.ds(s * B_PER_TILE, B_PER_TILE), :])

    @pl.when(c == 0)
    def _():
        pltpu.sync_copy(idx_hbm.at[pl.ds(s * B_PER_TILE, B_PER_TILE)], ibuf)
        pltpu.async_copy(emb32_hbm.at[ibuf], selbuf, gsem0).wait()
        pltpu.sync_copy(selbuf,
                        sel_out.at[2, pl.ds(s * B_PER_TILE, B_PER_TILE), :])


@functools.cache
def _make_sc_agg():
    return functools.partial(
        pl.kernel,
        out_type=(
            jax.ShapeDtypeStruct((3, BATCH, HIDDEN), jnp.float32),
            jax.ShapeDtypeStruct((2, BATCH, MSG_DIM), jnp.float32),
        ),
        mesh=plsc.VectorSubcoreMesh(core_axis_name="c", subcore_axis_name="s",
                                    num_cores=NC, num_subcores=NS),
        compiler_params=_SC_PARAMS,
        scratch_types=[
            pltpu.VMEM_SHARED((N_ACC, HIDDEN), jnp.float32),
            pltpu.VMEM_SHARED((N_ACC, MSG_DIM), jnp.float32),
            pltpu.VMEM((CAP,), jnp.int32),
            pltpu.VMEM((CAP,), jnp.int32),
            pltpu.VMEM((CAP,), jnp.int32),
            pltpu.VMEM((CHUNK,), jnp.int32),
            pltpu.VMEM((16,), jnp.int32),
            pltpu.VMEM((B_PER_TILE,), jnp.int32),
            pltpu.VMEM((B_PER_TILE,), jnp.int32),
            pltpu.VMEM((CHUNK, HIDDEN // 2), jnp.int32),
            pltpu.VMEM((CHUNK, HIDDEN // 2), jnp.int32),
            pltpu.VMEM((CHUNK, MSG_DIM), jnp.float32),
            pltpu.VMEM((CHUNK, HIDDEN // 2), jnp.int32),
            pltpu.VMEM((CHUNK, HIDDEN // 2), jnp.int32),
            pltpu.VMEM((CHUNK, MSG_DIM), jnp.float32),
            pltpu.VMEM((CHUNK, HIDDEN), jnp.float32),
            pltpu.VMEM((B_PER_TILE, HIDDEN), jnp.float32),
            pltpu.VMEM((B_PER_TILE, MSG_DIM), jnp.float32),
            pltpu.SemaphoreType.DMA,
            pltpu.SemaphoreType.DMA,
            pltpu.SemaphoreType.DMA,
            pltpu.SemaphoreType.DMA,
            pltpu.SemaphoreType.DMA,
            pltpu.SemaphoreType.DMA,
        ],
    )(_sc_agg_body)


def _cls_body(sel_ref, msel_ref, wm_ref, wu_ref, bu_ref, w1_ref, b1_ref,
              w2_ref, b2_ref, out_ref):
    x = (sel_ref[0] + sel_ref[1] + sel_ref[2]
         + jnp.dot(msel_ref[0] + msel_ref[1], wm_ref[...],
                   preferred_element_type=jnp.float32))
    h = jnp.maximum(
        jnp.dot(x, wu_ref[...], preferred_element_type=jnp.float32)
        + bu_ref[...], 0.0)
    h2 = jnp.maximum(
        jnp.dot(h, w1_ref[...], preferred_element_type=jnp.float32)
        + b1_ref[...], 0.0)
    out_ref[...] = (jnp.dot(h2, w2_ref[...],
                            preferred_element_type=jnp.float32)
                    + b2_ref[...])


def _classifier(sel, msel, W_msg, W_upd, bu, W1, b1, W2, b2):
    return pl.pallas_call(
        _cls_body,
        out_shape=jax.ShapeDtypeStruct((BATCH, HIDDEN), jnp.float32),
    )(sel, msel, W_msg, W_upd, bu, W1, b1, W2, b2)


def kernel(src, dst, t, msg, labels, idx, node_emb, w_time, W_msg, b_msg,
           W_upd, b_upd, W1, b1, W2, b2):
    del labels
    srcc, slotc, eidc, tcc, cnts, slotsel = _make_sc_compact()(
        src.reshape(NW, E_PER_W),
        dst.reshape(NW, E_PER_W),
        t.reshape(NW, E_PER_W),
        idx,
        jnp.full((N_NODES,), TRASH, jnp.int32))
    F = _cos_features(cnts[:, 0], tcc.reshape(NW * REG, 1), w_time,
                      b_msg.reshape(1, HIDDEN))
    eb = node_emb.astype(jnp.bfloat16)
    lo = (jax.lax.bitcast_convert_type(eb[:, :64], jnp.int16)
          .astype(jnp.int32) & 0xFFFF)
    hi = jax.lax.bitcast_convert_type(eb[:, 64:], jnp.int16).astype(
        jnp.int32) << 16
    emb_sw = hi | lo
    sel, msel = _make_sc_agg()(
        emb_sw, F, msg, node_emb, srcc, slotc, eidc, cnts, slotsel, idx,
        jnp.zeros((N_ACC, HIDDEN), jnp.float32),
        jnp.zeros((N_ACC, MSG_DIM), jnp.float32))
    return _classifier(sel, msel, W_msg, W_upd, b_upd.reshape(1, HIDDEN),
                       W1, b1.reshape(1, HIDDEN), W2, b2.reshape(1, HIDDEN))


# speedup vs baseline: 7.7279x; 1.3342x over previous
import functools

import jax
import jax.numpy as jnp
from jax import lax
from jax.experimental import pallas as pl
from jax.experimental.pallas import tpu as pltpu
from jax.experimental.pallas import tpu_sc as plsc

N_NODES = 10000
N_EDGES = 320000
HIDDEN = 128
MSG_DIM = 16
BATCH = 2048
T_MAX = 1000.0

NC, NS = 2, 16
NW = NC * NS
E_PER_W = N_EDGES // NW
CHUNK = 80
PADC = 2 * CHUNK
CAP = E_PER_W + PADC
B_PER_TILE = BATCH // NS
TRASH = BATCH
N_ACC = 2176
ROWS_PER_TILE = N_ACC // NS

BE = 1024
REG = 10240
KBLK = REG // BE

_SC_PARAMS = pltpu.CompilerParams(use_tc_tiling_on_sc=False,
                                  needs_layout_passes=False)


def _sc_compact_body(src_hbm, dst_hbm, t_hbm, idx_hbm, inv_hbm,
                     srcc_hbm, slotc_hbm, eidc_hbm, tcc_hbm, cnts_hbm,
                     slotsel_hbm,
                     slotmap, srcv, dstv, tvv, eidv, idxall, cntbuf, sselbuf):
    c = lax.axis_index("c")
    s = lax.axis_index("s")
    wid = c * NS + s
    pltpu.sync_copy(inv_hbm, slotmap)
    pltpu.sync_copy(idx_hbm, idxall)
    lanes = jnp.arange(16, dtype=jnp.int32)

    def sbody(k, carry):
        vi = idxall[pl.ds(k * 16, 16)]
        plsc.store_scatter(slotmap, [vi], lanes + k * 16)
        return carry

    lax.fori_loop(0, BATCH // 16, sbody, 0)
    base_e = wid * E_PER_W
    pltpu.sync_copy(src_hbm.at[pl.ds(base_e, E_PER_W)],
                    srcv.at[pl.ds(0, E_PER_W)])
    pltpu.sync_copy(dst_hbm.at[pl.ds(base_e, E_PER_W)],
                    dstv.at[pl.ds(0, E_PER_W)])
    pltpu.sync_copy(t_hbm.at[pl.ds(base_e, E_PER_W)],
                    tvv.at[pl.ds(0, E_PER_W)])

    def cbody(i, off):
        d = dstv[pl.ds(i * 16, 16)]
        sv = srcv[pl.ds(i * 16, 16)]
        tval = tvv[pl.ds(i * 16, 16)]
        slot = plsc.load_gather(slotmap, [d])
        m = (slot != TRASH) & (tval <= T_MAX)
        plsc.store_compressed(dstv.at[pl.ds(off, 16)], slot, mask=m)
        plsc.store_compressed(srcv.at[pl.ds(off, 16)], sv, mask=m)
        plsc.store_compressed(tvv.at[pl.ds(off, 16)], tval, mask=m)
        plsc.store_compressed(eidv.at[pl.ds(off, 16)],
                              lanes + (base_e + i * 16), mask=m)
        return off + jnp.sum(m.astype(jnp.int32))

    cnt = lax.fori_loop(0, E_PER_W // 16, cbody, jnp.int32(0))

    def pbody(g, carry):
        dstv[pl.ds(cnt + g * 16, 16)] = jnp.full((16,), TRASH, jnp.int32)
        srcv[pl.ds(cnt + g * 16, 16)] = jnp.zeros((16,), jnp.int32)
        tvv[pl.ds(cnt + g * 16, 16)] = jnp.zeros((16,), jnp.float32)
        eidv[pl.ds(cnt + g * 16, 16)] = jnp.zeros((16,), jnp.int32)
        return carry

    lax.fori_loop(0, PADC // 16, pbody, 0)

    pltpu.sync_copy(srcv, srcc_hbm.at[pl.ds(wid * REG, CAP)])
    pltpu.sync_copy(dstv, slotc_hbm.at[pl.ds(wid * REG, CAP)])
    pltpu.sync_copy(eidv, eidc_hbm.at[pl.ds(wid * REG, CAP)])
    pltpu.sync_copy(tvv, tcc_hbm.at[pl.ds(wid * REG, CAP)])
    cntbuf[pl.ds(0, 16)] = jnp.zeros((16,), jnp.int32) + cnt
    pltpu.sync_copy(cntbuf, cnts_hbm.at[pl.ds(wid * 16, 16)])

    for k in range(B_PER_TILE // 16):
        vi = idxall[pl.ds(s * B_PER_TILE + k * 16, 16)]
        sselbuf[pl.ds(k * 16, 16)] = plsc.load_gather(slotmap, [vi])

    @pl.when(c == 0)
    def _():
        pltpu.sync_copy(sselbuf,
                        slotsel_hbm.at[pl.ds(s * B_PER_TILE, B_PER_TILE)])


@functools.cache
def _make_sc_compact():
    return functools.partial(
        pl.kernel,
        out_type=(
            jax.ShapeDtypeStruct((NW * REG,), jnp.int32),
            jax.ShapeDtypeStruct((NW * REG,), jnp.int32),
            jax.ShapeDtypeStruct((NW * REG,), jnp.int32),
            jax.ShapeDtypeStruct((NW * REG,), jnp.float32),
            jax.ShapeDtypeStruct((NW * 16,), jnp.int32),
            jax.ShapeDtypeStruct((BATCH,), jnp.int32),
        ),
        mesh=plsc.VectorSubcoreMesh(core_axis_name="c", subcore_axis_name="s",
                                    num_cores=NC, num_subcores=NS),
        compiler_params=_SC_PARAMS,
        scratch_types=[
            pltpu.VMEM((N_NODES,), jnp.int32),
            pltpu.VMEM((CAP,), jnp.int32),
            pltpu.VMEM((CAP,), jnp.int32),
            pltpu.VMEM((CAP,), jnp.float32),
            pltpu.VMEM((CAP,), jnp.int32),
            pltpu.VMEM((BATCH,), jnp.int32),
            pltpu.VMEM((16,), jnp.int32),
            pltpu.VMEM((B_PER_TILE,), jnp.int32),
        ],
    )(_sc_compact_body)


def _rnb16(x):
    return (jax.lax.bitcast_convert_type(x, jnp.int32)
            + jnp.int32(0x8000)) & jnp.int32(-65536)


def _cos_body(cnts_ref, t_ref, w_ref, bm_ref, f_ref):
    g = pl.program_id(0)
    w = g // KBLK

    @pl.when((g % KBLK) * BE < cnts_ref[w * 16])
    def _():
        t = t_ref[...].reshape(BE, 1)
        f = jnp.cos(t * w_ref[...]) + bm_ref[...]
        lo = jax.lax.shift_right_logical(_rnb16(f[:, :64]), 16)
        f_ref[...] = _rnb16(f[:, 64:]) | lo


def _cos_features(cnts32, t_c, w_time, bm):
    return pl.pallas_call(
        _cos_body,
        grid_spec=pltpu.PrefetchScalarGridSpec(
            num_scalar_prefetch=1,
            grid=(NW * KBLK,),
            in_specs=[
                pl.BlockSpec((BE,), lambda g, cnts: (g,)),
                pl.BlockSpec((1, HIDDEN), lambda g, cnts: (0, 0)),
                pl.BlockSpec((1, HIDDEN), lambda g, cnts: (0, 0)),
            ],
            out_specs=pl.BlockSpec((BE, HIDDEN // 2), lambda g, cnts: (g, 0)),
        ),
        out_shape=jax.ShapeDtypeStruct((NW * REG, HIDDEN // 2), jnp.int32),
    )(cnts32, t_c, w_time, bm)


def _bf_lo(x):
    return jax.lax.bitcast_convert_type(jax.lax.shift_left(x, 16),
                                        jnp.float32)


def _bf_hi(x):
    return jax.lax.bitcast_convert_type(x & jnp.int32(-65536), jnp.float32)


def _sc_agg_body(embsw_hbm, f_hbm, msg_hbm, emb32_hbm, srcc_hbm, slotc_hbm,
                 eidc_hbm, cnts_hbm, slotsel_hbm, idx_hbm, zeros_hbm,
                 mzeros_hbm, sel_out, msel_out,
                 acc, macc, srcv, dstv, eidv, dstb, cntbuf, sselbuf, ibuf,
                 rows0, fbuf0, mrow0, rows1, fbuf1, mrow1, comb, selbuf,
                 mselbuf, gsem0, fsem0, msem0, gsem1, fsem1, msem1):
    c = lax.axis_index("c")
    s = lax.axis_index("s")
    wid = c * NS + s
    lanes = jnp.arange(16, dtype=jnp.int32)
    pltpu.sync_copy(zeros_hbm.at[pl.ds(s * ROWS_PER_TILE, ROWS_PER_TILE), :],
                    acc.at[pl.ds(s * ROWS_PER_TILE, ROWS_PER_TILE), :])
    pltpu.sync_copy(mzeros_hbm.at[pl.ds(s * ROWS_PER_TILE, ROWS_PER_TILE), :],
                    macc.at[pl.ds(s * ROWS_PER_TILE, ROWS_PER_TILE), :])
    pltpu.sync_copy(srcc_hbm.at[pl.ds(wid * REG, CAP)], srcv)
    pltpu.sync_copy(slotc_hbm.at[pl.ds(wid * REG, CAP)], dstv)
    pltpu.sync_copy(eidc_hbm.at[pl.ds(wid * REG, CAP)], eidv)
    pltpu.sync_copy(cnts_hbm.at[pl.ds(wid * 16, 16)], cntbuf)
    cnt = jnp.sum(jnp.where(lanes == 0, cntbuf[pl.ds(0, 16)], 0))
    plsc.subcore_barrier()

    fbase = wid * REG

    def start_loads(k, rows, fbuf, mrow, gsem, fsem, msem):
        pltpu.async_copy(embsw_hbm.at[srcv.at[pl.ds(k * CHUNK, CHUNK)]],
                         rows, gsem)
        pltpu.async_copy(f_hbm.at[pl.ds(fbase + k * CHUNK, CHUNK), :],
                         fbuf, fsem)
        pltpu.async_copy(msg_hbm.at[eidv.at[pl.ds(k * CHUNK, CHUNK)]],
                         mrow, msem)

    def wait_loads(k, rows, fbuf, mrow, gsem, fsem, msem):
        pltpu.make_async_copy(embsw_hbm.at[srcv.at[pl.ds(k * CHUNK, CHUNK)]],
                              rows, gsem).wait()
        pltpu.make_async_copy(f_hbm.at[pl.ds(fbase + k * CHUNK, CHUNK), :],
                              fbuf, fsem).wait()
        pltpu.make_async_copy(msg_hbm.at[eidv.at[pl.ds(k * CHUNK, CHUNK)]],
                              mrow, msem).wait()

    def scatter(k, rows, fbuf, mrow):
        for v in range(CHUNK // 16):
            dstb[pl.ds(v * 16, 16)] = dstv[pl.ds(k * CHUNK + v * 16, 16)]

        def gbody(i, carry):
            r = i // (HIDDEN // 32)
            gq = i % (HIDDEN // 32)
            xr = rows[r, pl.ds(gq * 16, 16)]
            xf = fbuf[r, pl.ds(gq * 16, 16)]
            comb[r, pl.ds(gq * 16, 16)] = _bf_lo(xr) + _bf_lo(xf)
            comb[r, pl.ds(64 + gq * 16, 16)] = _bf_hi(xr) + _bf_hi(xf)
            return carry

        lax.fori_loop(0, CHUNK * (HIDDEN // 32), gbody, 0)
        pltpu.sync_copy(comb, acc.at[dstb], add=True)
        pltpu.sync_copy(mrow, macc.at[dstb], add=True)

    npairs = (cnt + CHUNK - 1) // CHUNK // 2

    start_loads(0, rows0, fbuf0, mrow0, gsem0, fsem0, msem0)

    def body(j, carry):
        a = 2 * j
        start_loads(a + 1, rows1, fbuf1, mrow1, gsem1, fsem1, msem1)
        wait_loads(a, rows0, fbuf0, mrow0, gsem0, fsem0, msem0)
        scatter(a, rows0, fbuf0, mrow0)
        start_loads(a + 2, rows0, fbuf0, mrow0, gsem0, fsem0, msem0)
        wait_loads(a + 1, rows1, fbuf1, mrow1, gsem1, fsem1, msem1)
        scatter(a + 1, rows1, fbuf1, mrow1)
        return carry

    lax.fori_loop(0, npairs, body, 0)
    wait_loads(2 * npairs, rows0, fbuf0, mrow0, gsem0, fsem0, msem0)
    scatter(2 * npairs, rows0, fbuf0, mrow0)
    plsc.subcore_barrier()

    pltpu.sync_copy(slotsel_hbm.at[pl.ds(s * B_PER_TILE, B_PER_TILE)],
                    sselbuf)
    pltpu.sync_copy(acc.at[sselbuf], selbuf)
    pltpu.sync_copy(selbuf, sel_out.at[c, pl.ds(s * B_PER_TILE, B_PER_TILE), :])
    pltpu.sync_copy(macc.at[sselbuf], mselbuf)
    pltpu.sync_copy(mselbuf,
                    msel_out.at[c, pl.ds(s * B_PER_TILE, B_PER_TILE), :])

    @pl.when(c == 0)
    def _():
        pltpu.sync_copy(idx_hbm.at[pl.ds(s * B_PER_TILE, B_PER_TILE)], ibuf)
        pltpu.async_copy(emb32_hbm.at[ibuf], selbuf, gsem0).wait()
        pltpu.sync_copy(selbuf,
                        sel_out.at[2, pl.ds(s * B_PER_TILE, B_PER_TILE), :])


@functools.cache
def _make_sc_agg():
    return functools.partial(
        pl.kernel,
        out_type=(
            jax.ShapeDtypeStruct((3, BATCH, HIDDEN), jnp.float32),
            jax.ShapeDtypeStruct((2, BATCH, MSG_DIM), jnp.float32),
        ),
        mesh=plsc.VectorSubcoreMesh(core_axis_name="c", subcore_axis_name="s",
                                    num_cores=NC, num_subcores=NS),
        compiler_params=_SC_PARAMS,
        scratch_types=[
            pltpu.VMEM_SHARED((N_ACC, HIDDEN), jnp.float32),
            pltpu.VMEM_SHARED((N_ACC, MSG_DIM), jnp.float32),
            pltpu.VMEM((CAP,), jnp.int32),
            pltpu.VMEM((CAP,), jnp.int32),
            pltpu.VMEM((CAP,), jnp.int32),
            pltpu.VMEM((CHUNK,), jnp.int32),
            pltpu.VMEM((16,), jnp.int32),
            pltpu.VMEM((B_PER_TILE,), jnp.int32),
            pltpu.VMEM((B_PER_TILE,), jnp.int32),
            pltpu.VMEM((CHUNK, HIDDEN // 2), jnp.int32),
            pltpu.VMEM((CHUNK, HIDDEN // 2), jnp.int32),
            pltpu.VMEM((CHUNK, MSG_DIM), jnp.float32),
            pltpu.VMEM((CHUNK, HIDDEN // 2), jnp.int32),
            pltpu.VMEM((CHUNK, HIDDEN // 2), jnp.int32),
            pltpu.VMEM((CHUNK, MSG_DIM), jnp.float32),
            pltpu.VMEM((CHUNK, HIDDEN), jnp.float32),
            pltpu.VMEM((B_PER_TILE, HIDDEN), jnp.float32),
            pltpu.VMEM((B_PER_TILE, MSG_DIM), jnp.float32),
            pltpu.SemaphoreType.DMA,
            pltpu.SemaphoreType.DMA,
            pltpu.SemaphoreType.DMA,
            pltpu.SemaphoreType.DMA,
            pltpu.SemaphoreType.DMA,
            pltpu.SemaphoreType.DMA,
        ],
    )(_sc_agg_body)


def _cls_body(sel_ref, msel_ref, wm_ref, wu_ref, bu_ref, w1_ref, b1_ref,
              w2_ref, b2_ref, out_ref):
    x = (sel_ref[0] + sel_ref[1] + sel_ref[2]
         + jnp.dot(msel_ref[0] + msel_ref[1], wm_ref[...],
                   preferred_element_type=jnp.float32))
    h = jnp.maximum(
        jnp.dot(x, wu_ref[...], preferred_element_type=jnp.float32)
        + bu_ref[...], 0.0)
    h2 = jnp.maximum(
        jnp.dot(h, w1_ref[...], preferred_element_type=jnp.float32)
        + b1_ref[...], 0.0)
    out_ref[...] = (jnp.dot(h2, w2_ref[...],
                            preferred_element_type=jnp.float32)
                    + b2_ref[...])


def _classifier(sel, msel, W_msg, W_upd, bu, W1, b1, W2, b2):
    return pl.pallas_call(
        _cls_body,
        out_shape=jax.ShapeDtypeStruct((BATCH, HIDDEN), jnp.float32),
    )(sel, msel, W_msg, W_upd, bu, W1, b1, W2, b2)


def kernel(src, dst, t, msg, labels, idx, node_emb, w_time, W_msg, b_msg,
           W_upd, b_upd, W1, b1, W2, b2):
    del labels
    srcc, slotc, eidc, tcc, cnts, slotsel = _make_sc_compact()(
        src, dst, t, idx, jnp.full((N_NODES,), TRASH, jnp.int32))
    F = _cos_features(cnts, tcc, w_time, b_msg.reshape(1, HIDDEN))
    eb = node_emb.astype(jnp.bfloat16)
    lo = (jax.lax.bitcast_convert_type(eb[:, :64], jnp.int16)
          .astype(jnp.int32) & 0xFFFF)
    hi = jax.lax.bitcast_convert_type(eb[:, 64:], jnp.int16).astype(
        jnp.int32) << 16
    emb_sw = hi | lo
    sel, msel = _make_sc_agg()(
        emb_sw, F, msg, node_emb, srcc, slotc, eidc, cnts, slotsel, idx,
        jnp.zeros((N_ACC, HIDDEN), jnp.float32),
        jnp.zeros((N_ACC, MSG_DIM), jnp.float32))
    return _classifier(sel, msel, W_msg, W_upd, b_upd.reshape(1, HIDDEN),
                       W1, b1.reshape(1, HIDDEN), W2, b2.reshape(1, HIDDEN))


# speedup vs baseline: 10.0470x; 1.3001x over previous
import functools

import jax
import jax.numpy as jnp
from jax import lax
from jax.experimental import pallas as pl
from jax.experimental.pallas import tpu as pltpu
from jax.experimental.pallas import tpu_sc as plsc

N_NODES = 10000
N_EDGES = 320000
HIDDEN = 128
MSG_DIM = 16
BATCH = 2048
T_MAX = 1000.0

NC, NS = 2, 16
NW = NC * NS
E_PER_W = N_EDGES // NW
CHUNK = 64
PADC = 2 * CHUNK
CAP = E_PER_W + PADC
B_PER_TILE = BATCH // NS
TRASH = BATCH
N_ACC = 2176
ROWS_PER_TILE = N_ACC // NS

BE = 1024
REG = 10240
KBLK = REG // BE

_SC_PARAMS = pltpu.CompilerParams(use_tc_tiling_on_sc=False,
                                  needs_layout_passes=False)


def _sc_compact_body(src_hbm, dst_hbm, t_hbm, idx_hbm, inv_hbm,
                     srcc_hbm, slotc_hbm, eidc_hbm, tcc_hbm, cnts_hbm,
                     slotsel_hbm,
                     slotmap, srcv, dstv, tvv, eidv, idxall, cntbuf, sselbuf):
    c = lax.axis_index("c")
    s = lax.axis_index("s")
    wid = c * NS + s
    pltpu.sync_copy(inv_hbm, slotmap)
    pltpu.sync_copy(idx_hbm, idxall)
    lanes = jnp.arange(16, dtype=jnp.int32)

    def sbody(k, carry):
        vi = idxall[pl.ds(k * 16, 16)]
        plsc.store_scatter(slotmap, [vi], lanes + k * 16)
        return carry

    lax.fori_loop(0, BATCH // 16, sbody, 0)
    base_e = wid * E_PER_W
    pltpu.sync_copy(src_hbm.at[pl.ds(base_e, E_PER_W)],
                    srcv.at[pl.ds(0, E_PER_W)])
    pltpu.sync_copy(dst_hbm.at[pl.ds(base_e, E_PER_W)],
                    dstv.at[pl.ds(0, E_PER_W)])
    pltpu.sync_copy(t_hbm.at[pl.ds(base_e, E_PER_W)],
                    tvv.at[pl.ds(0, E_PER_W)])

    def cbody(i, off):
        d = dstv[pl.ds(i * 16, 16)]
        sv = srcv[pl.ds(i * 16, 16)]
        tval = tvv[pl.ds(i * 16, 16)]
        slot = plsc.load_gather(slotmap, [d])
        m = (slot != TRASH) & (tval <= T_MAX)
        plsc.store_compressed(dstv.at[pl.ds(off, 16)], slot, mask=m)
        plsc.store_compressed(srcv.at[pl.ds(off, 16)], sv, mask=m)
        plsc.store_compressed(tvv.at[pl.ds(off, 16)], tval, mask=m)
        plsc.store_compressed(eidv.at[pl.ds(off, 16)],
                              lanes + (base_e + i * 16), mask=m)
        return off + jnp.sum(m.astype(jnp.int32))

    cnt = lax.fori_loop(0, E_PER_W // 16, cbody, jnp.int32(0))

    def pbody(g, carry):
        dstv[pl.ds(cnt + g * 16, 16)] = jnp.full((16,), TRASH, jnp.int32)
        srcv[pl.ds(cnt + g * 16, 16)] = jnp.zeros((16,), jnp.int32)
        tvv[pl.ds(cnt + g * 16, 16)] = jnp.zeros((16,), jnp.float32)
        eidv[pl.ds(cnt + g * 16, 16)] = jnp.zeros((16,), jnp.int32)
        return carry

    lax.fori_loop(0, PADC // 16, pbody, 0)

    pltpu.sync_copy(srcv, srcc_hbm.at[pl.ds(wid * REG, CAP)])
    pltpu.sync_copy(dstv, slotc_hbm.at[pl.ds(wid * REG, CAP)])
    pltpu.sync_copy(eidv, eidc_hbm.at[pl.ds(wid * REG, CAP)])
    pltpu.sync_copy(tvv, tcc_hbm.at[pl.ds(wid * REG, CAP)])
    cntbuf[pl.ds(0, 16)] = jnp.zeros((16,), jnp.int32) + cnt
    pltpu.sync_copy(cntbuf, cnts_hbm.at[pl.ds(wid * 16, 16)])

    for k in range(B_PER_TILE // 16):
        vi = idxall[pl.ds(s * B_PER_TILE + k * 16, 16)]
        sselbuf[pl.ds(k * 16, 16)] = plsc.load_gather(slotmap, [vi])

    @pl.when(c == 0)
    def _():
        pltpu.sync_copy(sselbuf,
                        slotsel_hbm.at[pl.ds(s * B_PER_TILE, B_PER_TILE)])


@functools.cache
def _make_sc_compact():
    return functools.partial(
        pl.kernel,
        out_type=(
            jax.ShapeDtypeStruct((NW * REG,), jnp.int32),
            jax.ShapeDtypeStruct((NW * REG,), jnp.int32),
            jax.ShapeDtypeStruct((NW * REG,), jnp.int32),
            jax.ShapeDtypeStruct((NW * REG,), jnp.float32),
            jax.ShapeDtypeStruct((NW * 16,), jnp.int32),
            jax.ShapeDtypeStruct((BATCH,), jnp.int32),
        ),
        mesh=plsc.VectorSubcoreMesh(core_axis_name="c", subcore_axis_name="s",
                                    num_cores=NC, num_subcores=NS),
        compiler_params=_SC_PARAMS,
        scratch_types=[
            pltpu.VMEM((N_NODES,), jnp.int32),
            pltpu.VMEM((CAP,), jnp.int32),
            pltpu.VMEM((CAP,), jnp.int32),
            pltpu.VMEM((CAP,), jnp.float32),
            pltpu.VMEM((CAP,), jnp.int32),
            pltpu.VMEM((BATCH,), jnp.int32),
            pltpu.VMEM((16,), jnp.int32),
            pltpu.VMEM((B_PER_TILE,), jnp.int32),
        ],
    )(_sc_compact_body)


def _rnb16(x):
    return (jax.lax.bitcast_convert_type(x, jnp.int32)
            + jnp.int32(0x8000)) & jnp.int32(-65536)


def _cos_body(cnts_ref, t_ref, w_ref, bm_ref, f_ref):
    g = pl.program_id(0)
    w = g // KBLK

    @pl.when((g % KBLK) * BE < cnts_ref[w * 16])
    def _():
        t = t_ref[...].reshape(BE, 1)
        f = jnp.cos(t * w_ref[...]) + bm_ref[...]
        lo = jax.lax.shift_right_logical(_rnb16(f[:, :64]), 16)
        packed = _rnb16(f[:, 64:]) | lo
        f_ref[:, 0:64] = packed[:BE // 2]
        f_ref[:, 64:128] = packed[BE // 2:]


def _cos_features(cnts32, t_c, w_time, bm):
    return pl.pallas_call(
        _cos_body,
        grid_spec=pltpu.PrefetchScalarGridSpec(
            num_scalar_prefetch=1,
            grid=(NW * KBLK,),
            in_specs=[
                pl.BlockSpec((BE,), lambda g, cnts: (g,)),
                pl.BlockSpec((1, HIDDEN), lambda g, cnts: (0, 0)),
                pl.BlockSpec((1, HIDDEN), lambda g, cnts: (0, 0)),
            ],
            out_specs=pl.BlockSpec((BE // 2, HIDDEN),
                                   lambda g, cnts: (g, 0)),
        ),
        out_shape=jax.ShapeDtypeStruct((NW * REG // 2, HIDDEN), jnp.int32),
    )(cnts32, t_c, w_time, bm)


def _bf_lo(x):
    return jax.lax.bitcast_convert_type(jax.lax.shift_left(x, 16),
                                        jnp.float32)


def _bf_hi(x):
    return jax.lax.bitcast_convert_type(x & jnp.int32(-65536), jnp.float32)


def _sc_agg_body(embsw_hbm, f_hbm, msg_hbm, emb32_hbm, srcc_hbm, slotc_hbm,
                 eidc_hbm, cnts_hbm, slotsel_hbm, idx_hbm, zeros_hbm,
                 mzeros_hbm, sel_out, msel_out,
                 acc, macc, srcv, dstv, eidv, dstb, cntbuf, sselbuf, ibuf,
                 rows0, fbuf0, mrow0, rows1, fbuf1, mrow1, comb, selbuf,
                 mselbuf, gsem0, fsem0, msem0, gsem1, fsem1, msem1):
    c = lax.axis_index("c")
    s = lax.axis_index("s")
    wid = c * NS + s
    lanes = jnp.arange(16, dtype=jnp.int32)
    pltpu.sync_copy(zeros_hbm.at[pl.ds(s * ROWS_PER_TILE, ROWS_PER_TILE), :],
                    acc.at[pl.ds(s * ROWS_PER_TILE, ROWS_PER_TILE), :])
    pltpu.sync_copy(mzeros_hbm.at[pl.ds(s * ROWS_PER_TILE, ROWS_PER_TILE), :],
                    macc.at[pl.ds(s * ROWS_PER_TILE, ROWS_PER_TILE), :])
    pltpu.sync_copy(srcc_hbm.at[pl.ds(wid * REG, CAP)], srcv)
    pltpu.sync_copy(slotc_hbm.at[pl.ds(wid * REG, CAP)], dstv)
    pltpu.sync_copy(eidc_hbm.at[pl.ds(wid * REG, CAP)], eidv)
    pltpu.sync_copy(cnts_hbm.at[pl.ds(wid * 16, 16)], cntbuf)
    cnt = jnp.sum(jnp.where(lanes == 0, cntbuf[pl.ds(0, 16)], 0))
    plsc.subcore_barrier()

    fbase = wid * (REG // 2)

    def _fslice(k):
        p0 = k * CHUNK
        blk = p0 // BE
        within = p0 % BE
        h = within // (BE // 2)
        r0 = fbase + blk * (BE // 2) + within % (BE // 2)
        return f_hbm.at[pl.ds(r0, CHUNK), pl.ds(h * 64, 64)]

    def start_loads(k, rows, fbuf, mrow, gsem, fsem, msem):
        pltpu.async_copy(embsw_hbm.at[srcv.at[pl.ds(k * CHUNK, CHUNK)]],
                         rows, gsem)
        pltpu.async_copy(_fslice(k), fbuf, fsem)
        pltpu.async_copy(msg_hbm.at[eidv.at[pl.ds(k * CHUNK, CHUNK)]],
                         mrow, msem)

    def wait_loads(k, rows, fbuf, mrow, gsem, fsem, msem):
        pltpu.make_async_copy(embsw_hbm.at[srcv.at[pl.ds(k * CHUNK, CHUNK)]],
                              rows, gsem).wait()
        pltpu.make_async_copy(_fslice(k), fbuf, fsem).wait()
        pltpu.make_async_copy(msg_hbm.at[eidv.at[pl.ds(k * CHUNK, CHUNK)]],
                              mrow, msem).wait()

    def scatter(k, rows, fbuf, mrow):
        for v in range(CHUNK // 16):
            dstb[pl.ds(v * 16, 16)] = dstv[pl.ds(k * CHUNK + v * 16, 16)]

        def gbody(i, carry):
            r = i // (HIDDEN // 32)
            gq = i % (HIDDEN // 32)
            xr = rows[r, pl.ds(gq * 16, 16)]
            xf = fbuf[r, pl.ds(gq * 16, 16)]
            comb[r, pl.ds(gq * 16, 16)] = _bf_lo(xr) + _bf_lo(xf)
            comb[r, pl.ds(64 + gq * 16, 16)] = _bf_hi(xr) + _bf_hi(xf)
            return carry

        lax.fori_loop(0, CHUNK * (HIDDEN // 32), gbody, 0)
        pltpu.sync_copy(comb, acc.at[dstb], add=True)
        pltpu.sync_copy(mrow, macc.at[dstb], add=True)

    npairs = (cnt + CHUNK - 1) // CHUNK // 2

    start_loads(0, rows0, fbuf0, mrow0, gsem0, fsem0, msem0)

    def body(j, carry):
        a = 2 * j
        start_loads(a + 1, rows1, fbuf1, mrow1, gsem1, fsem1, msem1)
        wait_loads(a, rows0, fbuf0, mrow0, gsem0, fsem0, msem0)
        scatter(a, rows0, fbuf0, mrow0)
        start_loads(a + 2, rows0, fbuf0, mrow0, gsem0, fsem0, msem0)
        wait_loads(a + 1, rows1, fbuf1, mrow1, gsem1, fsem1, msem1)
        scatter(a + 1, rows1, fbuf1, mrow1)
        return carry

    lax.fori_loop(0, npairs, body, 0)
    wait_loads(2 * npairs, rows0, fbuf0, mrow0, gsem0, fsem0, msem0)
    scatter(2 * npairs, rows0, fbuf0, mrow0)
    plsc.subcore_barrier()

    pltpu.sync_copy(slotsel_hbm.at[pl.ds(s * B_PER_TILE, B_PER_TILE)],
                    sselbuf)
    pltpu.sync_copy(acc.at[sselbuf], selbuf)
    pltpu.sync_copy(selbuf, sel_out.at[c, pl.ds(s * B_PER_TILE, B_PER_TILE), :])
    pltpu.sync_copy(macc.at[sselbuf], mselbuf)
    pltpu.sync_copy(mselbuf,
                    msel_out.at[c, pl.ds(s * B_PER_TILE, B_PER_TILE), :])

    @pl.when(c == 0)
    def _():
        pltpu.sync_copy(idx_hbm.at[pl.ds(s * B_PER_TILE, B_PER_TILE)], ibuf)
        pltpu.async_copy(emb32_hbm.at[ibuf], selbuf, gsem0).wait()
        pltpu.sync_copy(selbuf,
                        sel_out.at[2, pl.ds(s * B_PER_TILE, B_PER_TILE), :])


@functools.cache
def _make_sc_agg():
    return functools.partial(
        pl.kernel,
        out_type=(
            jax.ShapeDtypeStruct((3, BATCH, HIDDEN), jnp.float32),
            jax.ShapeDtypeStruct((2, BATCH, MSG_DIM), jnp.float32),
        ),
        mesh=plsc.VectorSubcoreMesh(core_axis_name="c", subcore_axis_name="s",
                                    num_cores=NC, num_subcores=NS),
        compiler_params=_SC_PARAMS,
        scratch_types=[
            pltpu.VMEM_SHARED((N_ACC, HIDDEN), jnp.float32),
            pltpu.VMEM_SHARED((N_ACC, MSG_DIM), jnp.float32),
            pltpu.VMEM((CAP,), jnp.int32),
            pltpu.VMEM((CAP,), jnp.int32),
            pltpu.VMEM((CAP,), jnp.int32),
            pltpu.VMEM((CHUNK,), jnp.int32),
            pltpu.VMEM((16,), jnp.int32),
            pltpu.VMEM((B_PER_TILE,), jnp.int32),
            pltpu.VMEM((B_PER_TILE,), jnp.int32),
            pltpu.VMEM((CHUNK, HIDDEN // 2), jnp.int32),
            pltpu.VMEM((CHUNK, HIDDEN // 2), jnp.int32),
            pltpu.VMEM((CHUNK, MSG_DIM), jnp.float32),
            pltpu.VMEM((CHUNK, HIDDEN // 2), jnp.int32),
            pltpu.VMEM((CHUNK, HIDDEN // 2), jnp.int32),
            pltpu.VMEM((CHUNK, MSG_DIM), jnp.float32),
            pltpu.VMEM((CHUNK, HIDDEN), jnp.float32),
            pltpu.VMEM((B_PER_TILE, HIDDEN), jnp.float32),
            pltpu.VMEM((B_PER_TILE, MSG_DIM), jnp.float32),
            pltpu.SemaphoreType.DMA,
            pltpu.SemaphoreType.DMA,
            pltpu.SemaphoreType.DMA,
            pltpu.SemaphoreType.DMA,
            pltpu.SemaphoreType.DMA,
            pltpu.SemaphoreType.DMA,
        ],
    )(_sc_agg_body)


def _cls_body(sel_ref, msel_ref, wm_ref, wu_ref, bu_ref, w1_ref, b1_ref,
              w2_ref, b2_ref, out_ref):
    x = (sel_ref[0] + sel_ref[1] + sel_ref[2]
         + jnp.dot(msel_ref[0] + msel_ref[1], wm_ref[...],
                   preferred_element_type=jnp.float32))
    h = jnp.maximum(
        jnp.dot(x, wu_ref[...], preferred_element_type=jnp.float32)
        + bu_ref[...], 0.0)
    h2 = jnp.maximum(
        jnp.dot(h, w1_ref[...], preferred_element_type=jnp.float32)
        + b1_ref[...], 0.0)
    out_ref[...] = (jnp.dot(h2, w2_ref[...],
                            preferred_element_type=jnp.float32)
                    + b2_ref[...])


def _classifier(sel, msel, W_msg, W_upd, bu, W1, b1, W2, b2):
    return pl.pallas_call(
        _cls_body,
        out_shape=jax.ShapeDtypeStruct((BATCH, HIDDEN), jnp.float32),
    )(sel, msel, W_msg, W_upd, bu, W1, b1, W2, b2)


def kernel(src, dst, t, msg, labels, idx, node_emb, w_time, W_msg, b_msg,
           W_upd, b_upd, W1, b1, W2, b2):
    del labels
    srcc, slotc, eidc, tcc, cnts, slotsel = _make_sc_compact()(
        src, dst, t, idx, jnp.full((N_NODES,), TRASH, jnp.int32))
    F = _cos_features(cnts, tcc, w_time, b_msg.reshape(1, HIDDEN))
    eb = node_emb.astype(jnp.bfloat16)
    lo = (jax.lax.bitcast_convert_type(eb[:, :64], jnp.int16)
          .astype(jnp.int32) & 0xFFFF)
    hi = jax.lax.bitcast_convert_type(eb[:, 64:], jnp.int16).astype(
        jnp.int32) << 16
    emb_sw = hi | lo
    sel, msel = _make_sc_agg()(
        emb_sw, F, msg, node_emb, srcc, slotc, eidc, cnts, slotsel, idx,
        jnp.zeros((N_ACC, HIDDEN), jnp.float32),
        jnp.zeros((N_ACC, MSG_DIM), jnp.float32))
    return _classifier(sel, msel, W_msg, W_upd, b_upd.reshape(1, HIDDEN),
                       W1, b1.reshape(1, HIDDEN), W2, b2.reshape(1, HIDDEN))


# speedup vs baseline: 12.2024x; 1.2145x over previous
import functools

import jax
import jax.numpy as jnp
from jax import lax
from jax.experimental import pallas as pl
from jax.experimental.pallas import tpu as pltpu
from jax.experimental.pallas import tpu_sc as plsc

N_NODES = 10000
N_EDGES = 320000
HIDDEN = 128
MSG_DIM = 16
BATCH = 2048
T_MAX = 1000.0

NC, NS = 2, 16
NW = NC * NS
E_PER_W = N_EDGES // NW
CHUNK = 64
PADC = 2 * CHUNK
CAP = E_PER_W + PADC
B_PER_TILE = BATCH // NS
TRASH = BATCH
N_ACC = 2176
ROWS_PER_TILE = N_ACC // NS

BE = 1024
REG = 10240
KBLK = REG // BE

_SC_PARAMS = pltpu.CompilerParams(use_tc_tiling_on_sc=False,
                                  needs_layout_passes=False)


def _sc_compact_body(src_hbm, dst_hbm, t_hbm, idx_hbm, inv_hbm,
                     srcc_hbm, slotc_hbm, eidc_hbm, tcc_hbm, cnts_hbm,
                     slotsel_hbm,
                     slotmap, srcv, dstv, tvv, eidv, idxall, cntbuf, sselbuf):
    c = lax.axis_index("c")
    s = lax.axis_index("s")
    wid = c * NS + s
    pltpu.sync_copy(inv_hbm, slotmap)
    pltpu.sync_copy(idx_hbm, idxall)
    lanes = jnp.arange(16, dtype=jnp.int32)

    def sbody(k, carry):
        vi = idxall[pl.ds(k * 16, 16)]
        plsc.store_scatter(slotmap, [vi], lanes + k * 16)
        return carry

    lax.fori_loop(0, BATCH // 16, sbody, 0)
    base_e = wid * E_PER_W
    pltpu.sync_copy(src_hbm.at[pl.ds(base_e, E_PER_W)],
                    srcv.at[pl.ds(0, E_PER_W)])
    pltpu.sync_copy(dst_hbm.at[pl.ds(base_e, E_PER_W)],
                    dstv.at[pl.ds(0, E_PER_W)])
    pltpu.sync_copy(t_hbm.at[pl.ds(base_e, E_PER_W)],
                    tvv.at[pl.ds(0, E_PER_W)])

    def cbody(i, off):
        d = dstv[pl.ds(i * 16, 16)]
        sv = srcv[pl.ds(i * 16, 16)]
        tval = tvv[pl.ds(i * 16, 16)]
        slot = plsc.load_gather(slotmap, [d])
        m = (slot != TRASH) & (tval <= T_MAX)
        plsc.store_compressed(dstv.at[pl.ds(off, 16)], slot, mask=m)
        plsc.store_compressed(srcv.at[pl.ds(off, 16)], sv, mask=m)
        plsc.store_compressed(tvv.at[pl.ds(off, 16)], tval, mask=m)
        plsc.store_compressed(eidv.at[pl.ds(off, 16)],
                              lanes + (base_e + i * 16), mask=m)
        return off + jnp.sum(m.astype(jnp.int32))

    cnt = lax.fori_loop(0, E_PER_W // 16, cbody, jnp.int32(0))

    def pbody(g, carry):
        dstv[pl.ds(cnt + g * 16, 16)] = jnp.full((16,), TRASH, jnp.int32)
        srcv[pl.ds(cnt + g * 16, 16)] = jnp.zeros((16,), jnp.int32)
        tvv[pl.ds(cnt + g * 16, 16)] = jnp.zeros((16,), jnp.float32)
        eidv[pl.ds(cnt + g * 16, 16)] = jnp.zeros((16,), jnp.int32)
        return carry

    lax.fori_loop(0, PADC // 16, pbody, 0)

    pltpu.sync_copy(srcv, srcc_hbm.at[pl.ds(wid * REG, CAP)])
    pltpu.sync_copy(dstv, slotc_hbm.at[pl.ds(wid * REG, CAP)])
    pltpu.sync_copy(eidv, eidc_hbm.at[pl.ds(wid * REG, CAP)])
    pltpu.sync_copy(tvv, tcc_hbm.at[pl.ds(wid * REG, CAP)])
    cntbuf[pl.ds(0, 16)] = jnp.zeros((16,), jnp.int32) + cnt
    pltpu.sync_copy(cntbuf, cnts_hbm.at[pl.ds(wid * 16, 16)])

    for k in range(B_PER_TILE // 16):
        vi = idxall[pl.ds(s * B_PER_TILE + k * 16, 16)]
        sselbuf[pl.ds(k * 16, 16)] = plsc.load_gather(slotmap, [vi])

    @pl.when(c == 0)
    def _():
        pltpu.sync_copy(sselbuf,
                        slotsel_hbm.at[pl.ds(s * B_PER_TILE, B_PER_TILE)])


@functools.cache
def _make_sc_compact():
    return functools.partial(
        pl.kernel,
        out_type=(
            jax.ShapeDtypeStruct((NW * REG,), jnp.int32),
            jax.ShapeDtypeStruct((NW * REG,), jnp.int32),
            jax.ShapeDtypeStruct((NW * REG,), jnp.int32),
            jax.ShapeDtypeStruct((NW * REG,), jnp.float32),
            jax.ShapeDtypeStruct((NW * 16,), jnp.int32),
            jax.ShapeDtypeStruct((BATCH,), jnp.int32),
        ),
        mesh=plsc.VectorSubcoreMesh(core_axis_name="c", subcore_axis_name="s",
                                    num_cores=NC, num_subcores=NS),
        compiler_params=_SC_PARAMS,
        scratch_types=[
            pltpu.VMEM((N_NODES,), jnp.int32),
            pltpu.VMEM((CAP,), jnp.int32),
            pltpu.VMEM((CAP,), jnp.int32),
            pltpu.VMEM((CAP,), jnp.float32),
            pltpu.VMEM((CAP,), jnp.int32),
            pltpu.VMEM((BATCH,), jnp.int32),
            pltpu.VMEM((16,), jnp.int32),
            pltpu.VMEM((B_PER_TILE,), jnp.int32),
        ],
    )(_sc_compact_body)


def _rnb16(x):
    return (jax.lax.bitcast_convert_type(x, jnp.int32)
            + jnp.int32(0x8000)) & jnp.int32(-65536)


def _cos_body(cnts_ref, t_ref, w_ref, bm_ref, f_ref):
    g = pl.program_id(0)
    w = g // KBLK

    @pl.when((g % KBLK) * BE < cnts_ref[w * 16])
    def _():
        t = t_ref[...].reshape(BE, 1)
        f = jnp.cos(t * w_ref[...]) + bm_ref[...]
        lo = jax.lax.shift_right_logical(_rnb16(f[:, :64]), 16)
        packed = _rnb16(f[:, 64:]) | lo
        f_ref[:, 0:64] = packed[:BE // 2]
        f_ref[:, 64:128] = packed[BE // 2:]


def _cos_features(cnts32, t_c, w_time, bm):
    def _blk(g, cnts):
        w = g // KBLK
        active = (g % KBLK) * BE < cnts[w * 16]
        return jnp.where(active, g, w * KBLK + KBLK - 1)

    return pl.pallas_call(
        _cos_body,
        grid_spec=pltpu.PrefetchScalarGridSpec(
            num_scalar_prefetch=1,
            grid=(NW * KBLK,),
            in_specs=[
                pl.BlockSpec((BE,), lambda g, cnts: (_blk(g, cnts),)),
                pl.BlockSpec((1, HIDDEN), lambda g, cnts: (0, 0)),
                pl.BlockSpec((1, HIDDEN), lambda g, cnts: (0, 0)),
            ],
            out_specs=pl.BlockSpec((BE // 2, HIDDEN),
                                   lambda g, cnts: (_blk(g, cnts), 0)),
        ),
        out_shape=jax.ShapeDtypeStruct((NW * REG // 2, HIDDEN), jnp.int32),
    )(cnts32, t_c, w_time, bm)


def _bf_lo(x):
    return jax.lax.bitcast_convert_type(jax.lax.shift_left(x, 16),
                                        jnp.float32)


def _bf_hi(x):
    return jax.lax.bitcast_convert_type(x & jnp.int32(-65536), jnp.float32)


def _sc_agg_body(embsw_hbm, f_hbm, msg_hbm, emb32_hbm, srcc_hbm, slotc_hbm,
                 eidc_hbm, cnts_hbm, slotsel_hbm, idx_hbm, zeros_hbm,
                 mzeros_hbm, sel_out, msel_out,
                 acc, macc, srcv, dstv, eidv, dstb, cntbuf, sselbuf, ibuf,
                 rows0, fbuf0, mrow0, rows1, fbuf1, mrow1, comb, selbuf,
                 mselbuf, gsem0, fsem0, msem0, gsem1, fsem1, msem1):
    c = lax.axis_index("c")
    s = lax.axis_index("s")
    wid = c * NS + s
    lanes = jnp.arange(16, dtype=jnp.int32)
    pltpu.sync_copy(zeros_hbm.at[pl.ds(s * ROWS_PER_TILE, ROWS_PER_TILE), :],
                    acc.at[pl.ds(s * ROWS_PER_TILE, ROWS_PER_TILE), :])
    pltpu.sync_copy(mzeros_hbm.at[pl.ds(s * ROWS_PER_TILE, ROWS_PER_TILE), :],
                    macc.at[pl.ds(s * ROWS_PER_TILE, ROWS_PER_TILE), :])
    pltpu.sync_copy(srcc_hbm.at[pl.ds(wid * REG, CAP)], srcv)
    pltpu.sync_copy(slotc_hbm.at[pl.ds(wid * REG, CAP)], dstv)
    pltpu.sync_copy(eidc_hbm.at[pl.ds(wid * REG, CAP)], eidv)
    pltpu.sync_copy(cnts_hbm.at[pl.ds(wid * 16, 16)], cntbuf)
    cnt = jnp.sum(jnp.where(lanes == 0, cntbuf[pl.ds(0, 16)], 0))
    plsc.subcore_barrier()

    fbase = wid * (REG // 2)

    def _fslice(k):
        p0 = k * CHUNK
        blk = p0 // BE
        within = p0 % BE
        h = within // (BE // 2)
        r0 = fbase + blk * (BE // 2) + within % (BE // 2)
        return f_hbm.at[pl.ds(r0, CHUNK), pl.ds(h * 64, 64)]

    def start_loads(k, rows, fbuf, mrow, gsem, fsem, msem):
        pltpu.async_copy(embsw_hbm.at[srcv.at[pl.ds(k * CHUNK, CHUNK)]],
                         rows, gsem)
        pltpu.async_copy(_fslice(k), fbuf, fsem)
        pltpu.async_copy(msg_hbm.at[eidv.at[pl.ds(k * CHUNK, CHUNK)]],
                         mrow, msem)

    def wait_loads(k, rows, fbuf, mrow, gsem, fsem, msem):
        pltpu.make_async_copy(embsw_hbm.at[srcv.at[pl.ds(k * CHUNK, CHUNK)]],
                              rows, gsem).wait()
        pltpu.make_async_copy(_fslice(k), fbuf, fsem).wait()
        pltpu.make_async_copy(msg_hbm.at[eidv.at[pl.ds(k * CHUNK, CHUNK)]],
                              mrow, msem).wait()

    def scatter(k, rows, fbuf, mrow):
        for v in range(CHUNK // 16):
            dstb[pl.ds(v * 16, 16)] = dstv[pl.ds(k * CHUNK + v * 16, 16)]

        def gbody(i, carry):
            r = i // (HIDDEN // 32)
            gq = i % (HIDDEN // 32)
            xr = rows[r, pl.ds(gq * 16, 16)]
            xf = fbuf[r, pl.ds(gq * 16, 16)]
            comb[r, pl.ds(gq * 16, 16)] = _bf_lo(xr) + _bf_lo(xf)
            comb[r, pl.ds(64 + gq * 16, 16)] = _bf_hi(xr) + _bf_hi(xf)
            return carry

        lax.fori_loop(0, CHUNK * (HIDDEN // 32), gbody, 0)
        pltpu.sync_copy(comb, acc.at[dstb], add=True)
        pltpu.sync_copy(mrow, macc.at[dstb], add=True)

    npairs = (cnt + CHUNK - 1) // CHUNK // 2

    start_loads(0, rows0, fbuf0, mrow0, gsem0, fsem0, msem0)

    def body(j, carry):
        a = 2 * j
        start_loads(a + 1, rows1, fbuf1, mrow1, gsem1, fsem1, msem1)
        wait_loads(a, rows0, fbuf0, mrow0, gsem0, fsem0, msem0)
        scatter(a, rows0, fbuf0, mrow0)
        start_loads(a + 2, rows0, fbuf0, mrow0, gsem0, fsem0, msem0)
        wait_loads(a + 1, rows1, fbuf1, mrow1, gsem1, fsem1, msem1)
        scatter(a + 1, rows1, fbuf1, mrow1)
        return carry

    lax.fori_loop(0, npairs, body, 0)
    wait_loads(2 * npairs, rows0, fbuf0, mrow0, gsem0, fsem0, msem0)
    scatter(2 * npairs, rows0, fbuf0, mrow0)
    plsc.subcore_barrier()

    pltpu.sync_copy(slotsel_hbm.at[pl.ds(s * B_PER_TILE, B_PER_TILE)],
                    sselbuf)
    pltpu.sync_copy(acc.at[sselbuf], selbuf)
    pltpu.sync_copy(selbuf, sel_out.at[c, pl.ds(s * B_PER_TILE, B_PER_TILE), :])
    pltpu.sync_copy(macc.at[sselbuf], mselbuf)
    pltpu.sync_copy(mselbuf,
                    msel_out.at[c, pl.ds(s * B_PER_TILE, B_PER_TILE), :])

    @pl.when(c == 0)
    def _():
        pltpu.sync_copy(idx_hbm.at[pl.ds(s * B_PER_TILE, B_PER_TILE)], ibuf)
        pltpu.async_copy(emb32_hbm.at[ibuf], selbuf, gsem0).wait()
        pltpu.sync_copy(selbuf,
                        sel_out.at[2, pl.ds(s * B_PER_TILE, B_PER_TILE), :])


@functools.cache
def _make_sc_agg():
    return functools.partial(
        pl.kernel,
        out_type=(
            jax.ShapeDtypeStruct((3, BATCH, HIDDEN), jnp.float32),
            jax.ShapeDtypeStruct((2, BATCH, MSG_DIM), jnp.float32),
        ),
        mesh=plsc.VectorSubcoreMesh(core_axis_name="c", subcore_axis_name="s",
                                    num_cores=NC, num_subcores=NS),
        compiler_params=_SC_PARAMS,
        scratch_types=[
            pltpu.VMEM_SHARED((N_ACC, HIDDEN), jnp.float32),
            pltpu.VMEM_SHARED((N_ACC, MSG_DIM), jnp.float32),
            pltpu.VMEM((CAP,), jnp.int32),
            pltpu.VMEM((CAP,), jnp.int32),
            pltpu.VMEM((CAP,), jnp.int32),
            pltpu.VMEM((CHUNK,), jnp.int32),
            pltpu.VMEM((16,), jnp.int32),
            pltpu.VMEM((B_PER_TILE,), jnp.int32),
            pltpu.VMEM((B_PER_TILE,), jnp.int32),
            pltpu.VMEM((CHUNK, HIDDEN // 2), jnp.int32),
            pltpu.VMEM((CHUNK, HIDDEN // 2), jnp.int32),
            pltpu.VMEM((CHUNK, MSG_DIM), jnp.float32),
            pltpu.VMEM((CHUNK, HIDDEN // 2), jnp.int32),
            pltpu.VMEM((CHUNK, HIDDEN // 2), jnp.int32),
            pltpu.VMEM((CHUNK, MSG_DIM), jnp.float32),
            pltpu.VMEM((CHUNK, HIDDEN), jnp.float32),
            pltpu.VMEM((B_PER_TILE, HIDDEN), jnp.float32),
            pltpu.VMEM((B_PER_TILE, MSG_DIM), jnp.float32),
            pltpu.SemaphoreType.DMA,
            pltpu.SemaphoreType.DMA,
            pltpu.SemaphoreType.DMA,
            pltpu.SemaphoreType.DMA,
            pltpu.SemaphoreType.DMA,
            pltpu.SemaphoreType.DMA,
        ],
    )(_sc_agg_body)


def _cls_body(sel_ref, msel_ref, wm_ref, wu_ref, bu_ref, w1_ref, b1_ref,
              w2_ref, b2_ref, out_ref):
    x = (sel_ref[0] + sel_ref[1] + sel_ref[2]
         + jnp.dot(msel_ref[0] + msel_ref[1], wm_ref[...],
                   preferred_element_type=jnp.float32))
    h = jnp.maximum(
        jnp.dot(x, wu_ref[...], preferred_element_type=jnp.float32)
        + bu_ref[...], 0.0)
    h2 = jnp.maximum(
        jnp.dot(h, w1_ref[...], preferred_element_type=jnp.float32)
        + b1_ref[...], 0.0)
    out_ref[...] = (jnp.dot(h2, w2_ref[...],
                            preferred_element_type=jnp.float32)
                    + b2_ref[...])


def _classifier(sel, msel, W_msg, W_upd, bu, W1, b1, W2, b2):
    return pl.pallas_call(
        _cls_body,
        out_shape=jax.ShapeDtypeStruct((BATCH, HIDDEN), jnp.float32),
    )(sel, msel, W_msg, W_upd, bu, W1, b1, W2, b2)


def kernel(src, dst, t, msg, labels, idx, node_emb, w_time, W_msg, b_msg,
           W_upd, b_upd, W1, b1, W2, b2):
    del labels
    srcc, slotc, eidc, tcc, cnts, slotsel = _make_sc_compact()(
        src, dst, t, idx, jnp.full((N_NODES,), TRASH, jnp.int32))
    F = _cos_features(cnts, tcc, w_time, b_msg.reshape(1, HIDDEN))
    eb = node_emb.astype(jnp.bfloat16)
    lo = (jax.lax.bitcast_convert_type(eb[:, :64], jnp.int16)
          .astype(jnp.int32) & 0xFFFF)
    hi = jax.lax.bitcast_convert_type(eb[:, 64:], jnp.int16).astype(
        jnp.int32) << 16
    emb_sw = hi | lo
    sel, msel = _make_sc_agg()(
        emb_sw, F, msg, node_emb, srcc, slotc, eidc, cnts, slotsel, idx,
        jnp.zeros((N_ACC, HIDDEN), jnp.float32),
        jnp.zeros((N_ACC, MSG_DIM), jnp.float32))
    return _classifier(sel, msel, W_msg, W_upd, b_upd.reshape(1, HIDDEN),
                       W1, b1.reshape(1, HIDDEN), W2, b2.reshape(1, HIDDEN))


# speedup vs baseline: 12.2665x; 1.0053x over previous
import functools

import jax
import jax.numpy as jnp
from jax import lax
from jax.experimental import pallas as pl
from jax.experimental.pallas import tpu as pltpu
from jax.experimental.pallas import tpu_sc as plsc

N_NODES = 10000
N_EDGES = 320000
HIDDEN = 128
MSG_DIM = 16
BATCH = 2048
T_MAX = 1000.0

NC, NS = 2, 16
NW = NC * NS
E_PER_W = N_EDGES // NW
CHUNK = 64
PADC = 2 * CHUNK
CAP = E_PER_W + PADC
B_PER_TILE = BATCH // NS
TRASH = BATCH
N_ACC = 2176
ROWS_PER_TILE = N_ACC // NS

BE = 2048
REG = 10240
KBLK = REG // BE

_SC_PARAMS = pltpu.CompilerParams(use_tc_tiling_on_sc=False,
                                  needs_layout_passes=False)


def _sc_compact_body(src_hbm, dst_hbm, t_hbm, idx_hbm, inv_hbm,
                     srcc_hbm, slotc_hbm, eidc_hbm, tcc_hbm, cnts_hbm,
                     slotsel_hbm,
                     slotmap, srcv, dstv, tvv, eidv, idxall, cntbuf, sselbuf):
    c = lax.axis_index("c")
    s = lax.axis_index("s")
    wid = c * NS + s
    pltpu.sync_copy(inv_hbm, slotmap)
    pltpu.sync_copy(idx_hbm, idxall)
    lanes = jnp.arange(16, dtype=jnp.int32)

    def sbody(k, carry):
        vi = idxall[pl.ds(k * 16, 16)]
        plsc.store_scatter(slotmap, [vi], lanes + k * 16)
        return carry

    lax.fori_loop(0, BATCH // 16, sbody, 0)
    base_e = wid * E_PER_W
    pltpu.sync_copy(src_hbm.at[pl.ds(base_e, E_PER_W)],
                    srcv.at[pl.ds(0, E_PER_W)])
    pltpu.sync_copy(dst_hbm.at[pl.ds(base_e, E_PER_W)],
                    dstv.at[pl.ds(0, E_PER_W)])
    pltpu.sync_copy(t_hbm.at[pl.ds(base_e, E_PER_W)],
                    tvv.at[pl.ds(0, E_PER_W)])

    def cbody(i, off):
        d = dstv[pl.ds(i * 16, 16)]
        sv = srcv[pl.ds(i * 16, 16)]
        tval = tvv[pl.ds(i * 16, 16)]
        slot = plsc.load_gather(slotmap, [d])
        m = (slot != TRASH) & (tval <= T_MAX)
        plsc.store_compressed(dstv.at[pl.ds(off, 16)], slot, mask=m)
        plsc.store_compressed(srcv.at[pl.ds(off, 16)], sv, mask=m)
        plsc.store_compressed(tvv.at[pl.ds(off, 16)], tval, mask=m)
        plsc.store_compressed(eidv.at[pl.ds(off, 16)],
                              lanes + (base_e + i * 16), mask=m)
        return off + jnp.sum(m.astype(jnp.int32))

    cnt = lax.fori_loop(0, E_PER_W // 16, cbody, jnp.int32(0))

    def pbody(g, carry):
        dstv[pl.ds(cnt + g * 16, 16)] = jnp.full((16,), TRASH, jnp.int32)
        srcv[pl.ds(cnt + g * 16, 16)] = jnp.zeros((16,), jnp.int32)
        tvv[pl.ds(cnt + g * 16, 16)] = jnp.zeros((16,), jnp.float32)
        eidv[pl.ds(cnt + g * 16, 16)] = jnp.zeros((16,), jnp.int32)
        return carry

    lax.fori_loop(0, PADC // 16, pbody, 0)

    pltpu.sync_copy(srcv, srcc_hbm.at[pl.ds(wid * REG, CAP)])
    pltpu.sync_copy(dstv, slotc_hbm.at[pl.ds(wid * REG, CAP)])
    pltpu.sync_copy(eidv, eidc_hbm.at[pl.ds(wid * REG, CAP)])
    pltpu.sync_copy(tvv, tcc_hbm.at[pl.ds(wid * REG, CAP)])
    cntbuf[pl.ds(0, 16)] = jnp.zeros((16,), jnp.int32) + cnt
    pltpu.sync_copy(cntbuf, cnts_hbm.at[pl.ds(wid * 16, 16)])

    for k in range(B_PER_TILE // 16):
        vi = idxall[pl.ds(s * B_PER_TILE + k * 16, 16)]
        sselbuf[pl.ds(k * 16, 16)] = plsc.load_gather(slotmap, [vi])

    @pl.when(c == 0)
    def _():
        pltpu.sync_copy(sselbuf,
                        slotsel_hbm.at[pl.ds(s * B_PER_TILE, B_PER_TILE)])


@functools.cache
def _make_sc_compact():
    return functools.partial(
        pl.kernel,
        out_type=(
            jax.ShapeDtypeStruct((NW * REG,), jnp.int32),
            jax.ShapeDtypeStruct((NW * REG,), jnp.int32),
            jax.ShapeDtypeStruct((NW * REG,), jnp.int32),
            jax.ShapeDtypeStruct((NW * REG,), jnp.float32),
            jax.ShapeDtypeStruct((NW * 16,), jnp.int32),
            jax.ShapeDtypeStruct((BATCH,), jnp.int32),
        ),
        mesh=plsc.VectorSubcoreMesh(core_axis_name="c", subcore_axis_name="s",
                                    num_cores=NC, num_subcores=NS),
        compiler_params=_SC_PARAMS,
        scratch_types=[
            pltpu.VMEM((N_NODES,), jnp.int32),
            pltpu.VMEM((CAP,), jnp.int32),
            pltpu.VMEM((CAP,), jnp.int32),
            pltpu.VMEM((CAP,), jnp.float32),
            pltpu.VMEM((CAP,), jnp.int32),
            pltpu.VMEM((BATCH,), jnp.int32),
            pltpu.VMEM((16,), jnp.int32),
            pltpu.VMEM((B_PER_TILE,), jnp.int32),
        ],
    )(_sc_compact_body)


def _rnb16(x):
    return (jax.lax.bitcast_convert_type(x, jnp.int32)
            + jnp.int32(0x8000)) & jnp.int32(-65536)


def _cos_body(cnts_ref, t_ref, w_ref, bm_ref, f_ref):
    g = pl.program_id(0)
    w = g // KBLK

    @pl.when((g % KBLK) * BE < cnts_ref[w * 16])
    def _():
        t = t_ref[...].reshape(BE, 1)
        f = jnp.cos(t * w_ref[...]) + bm_ref[...]
        lo = jax.lax.shift_right_logical(_rnb16(f[:, :64]), 16)
        packed = _rnb16(f[:, 64:]) | lo
        f_ref[:, 0:64] = packed[:BE // 2]
        f_ref[:, 64:128] = packed[BE // 2:]


def _cos_features(cnts32, t_c, w_time, bm):
    def _blk(g, cnts):
        w = g // KBLK
        active = (g % KBLK) * BE < cnts[w * 16]
        return jnp.where(active, g, w * KBLK + KBLK - 1)

    return pl.pallas_call(
        _cos_body,
        grid_spec=pltpu.PrefetchScalarGridSpec(
            num_scalar_prefetch=1,
            grid=(NW * KBLK,),
            in_specs=[
                pl.BlockSpec((BE,), lambda g, cnts: (_blk(g, cnts),)),
                pl.BlockSpec((1, HIDDEN), lambda g, cnts: (0, 0)),
                pl.BlockSpec((1, HIDDEN), lambda g, cnts: (0, 0)),
            ],
            out_specs=pl.BlockSpec((BE // 2, HIDDEN),
                                   lambda g, cnts: (_blk(g, cnts), 0)),
        ),
        out_shape=jax.ShapeDtypeStruct((NW * REG // 2, HIDDEN), jnp.int32),
    )(cnts32, t_c, w_time, bm)


def _bf_lo(x):
    return jax.lax.bitcast_convert_type(jax.lax.shift_left(x, 16),
                                        jnp.float32)


def _bf_hi(x):
    return jax.lax.bitcast_convert_type(x & jnp.int32(-65536), jnp.float32)


def _sc_agg_body(embsw_hbm, f_hbm, msg_hbm, emb32_hbm, srcc_hbm, slotc_hbm,
                 eidc_hbm, cnts_hbm, slotsel_hbm, idx_hbm, zeros_hbm,
                 mzeros_hbm, sel_out, msel_out,
                 acc, macc, srcv, dstv, eidv, dstb, cntbuf, sselbuf, ibuf,
                 rows0, fbuf0, mrow0, rows1, fbuf1, mrow1, comb, selbuf,
                 mselbuf, gsem0, fsem0, msem0, gsem1, fsem1, msem1):
    c = lax.axis_index("c")
    s = lax.axis_index("s")
    wid = c * NS + s
    lanes = jnp.arange(16, dtype=jnp.int32)
    pltpu.sync_copy(zeros_hbm.at[pl.ds(s * ROWS_PER_TILE, ROWS_PER_TILE), :],
                    acc.at[pl.ds(s * ROWS_PER_TILE, ROWS_PER_TILE), :])
    pltpu.sync_copy(mzeros_hbm.at[pl.ds(s * ROWS_PER_TILE, ROWS_PER_TILE), :],
                    macc.at[pl.ds(s * ROWS_PER_TILE, ROWS_PER_TILE), :])
    pltpu.sync_copy(srcc_hbm.at[pl.ds(wid * REG, CAP)], srcv)
    pltpu.sync_copy(slotc_hbm.at[pl.ds(wid * REG, CAP)], dstv)
    pltpu.sync_copy(eidc_hbm.at[pl.ds(wid * REG, CAP)], eidv)
    pltpu.sync_copy(cnts_hbm.at[pl.ds(wid * 16, 16)], cntbuf)
    cnt = jnp.sum(jnp.where(lanes == 0, cntbuf[pl.ds(0, 16)], 0))
    plsc.subcore_barrier()

    fbase = wid * (REG // 2)

    def _fslice(k):
        p0 = k * CHUNK
        blk = p0 // BE
        within = p0 % BE
        h = within // (BE // 2)
        r0 = fbase + blk * (BE // 2) + within % (BE // 2)
        return f_hbm.at[pl.ds(r0, CHUNK), pl.ds(h * 64, 64)]

    def start_loads(k, rows, fbuf, mrow, gsem, fsem, msem):
        pltpu.async_copy(embsw_hbm.at[srcv.at[pl.ds(k * CHUNK, CHUNK)]],
                         rows, gsem)
        pltpu.async_copy(_fslice(k), fbuf, fsem)
        pltpu.async_copy(msg_hbm.at[eidv.at[pl.ds(k * CHUNK, CHUNK)]],
                         mrow, msem)

    def wait_loads(k, rows, fbuf, mrow, gsem, fsem, msem):
        pltpu.make_async_copy(embsw_hbm.at[srcv.at[pl.ds(k * CHUNK, CHUNK)]],
                              rows, gsem).wait()
        pltpu.make_async_copy(_fslice(k), fbuf, fsem).wait()
        pltpu.make_async_copy(msg_hbm.at[eidv.at[pl.ds(k * CHUNK, CHUNK)]],
                              mrow, msem).wait()

    def scatter(k, rows, fbuf, mrow):
        for v in range(CHUNK // 16):
            dstb[pl.ds(v * 16, 16)] = dstv[pl.ds(k * CHUNK + v * 16, 16)]

        def gbody(i, carry):
            r = i // (HIDDEN // 32)
            gq = i % (HIDDEN // 32)
            xr = rows[r, pl.ds(gq * 16, 16)]
            xf = fbuf[r, pl.ds(gq * 16, 16)]
            comb[r, pl.ds(gq * 16, 16)] = _bf_lo(xr) + _bf_lo(xf)
            comb[r, pl.ds(64 + gq * 16, 16)] = _bf_hi(xr) + _bf_hi(xf)
            return carry

        lax.fori_loop(0, CHUNK * (HIDDEN // 32), gbody, 0)
        pltpu.sync_copy(comb, acc.at[dstb], add=True)
        pltpu.sync_copy(mrow, macc.at[dstb], add=True)

    npairs = (cnt + CHUNK - 1) // CHUNK // 2

    start_loads(0, rows0, fbuf0, mrow0, gsem0, fsem0, msem0)

    def body(j, carry):
        a = 2 * j
        start_loads(a + 1, rows1, fbuf1, mrow1, gsem1, fsem1, msem1)
        wait_loads(a, rows0, fbuf0, mrow0, gsem0, fsem0, msem0)
        scatter(a, rows0, fbuf0, mrow0)
        start_loads(a + 2, rows0, fbuf0, mrow0, gsem0, fsem0, msem0)
        wait_loads(a + 1, rows1, fbuf1, mrow1, gsem1, fsem1, msem1)
        scatter(a + 1, rows1, fbuf1, mrow1)
        return carry

    lax.fori_loop(0, npairs, body, 0)
    wait_loads(2 * npairs, rows0, fbuf0, mrow0, gsem0, fsem0, msem0)
    scatter(2 * npairs, rows0, fbuf0, mrow0)
    plsc.subcore_barrier()

    pltpu.sync_copy(slotsel_hbm.at[pl.ds(s * B_PER_TILE, B_PER_TILE)],
                    sselbuf)
    pltpu.sync_copy(acc.at[sselbuf], selbuf)
    pltpu.sync_copy(selbuf, sel_out.at[c, pl.ds(s * B_PER_TILE, B_PER_TILE), :])
    pltpu.sync_copy(macc.at[sselbuf], mselbuf)
    pltpu.sync_copy(mselbuf,
                    msel_out.at[c, pl.ds(s * B_PER_TILE, B_PER_TILE), :])

    @pl.when(c == 0)
    def _():
        pltpu.sync_copy(idx_hbm.at[pl.ds(s * B_PER_TILE, B_PER_TILE)], ibuf)
        pltpu.async_copy(emb32_hbm.at[ibuf], selbuf, gsem0).wait()
        pltpu.sync_copy(selbuf,
                        sel_out.at[2, pl.ds(s * B_PER_TILE, B_PER_TILE), :])


@functools.cache
def _make_sc_agg():
    return functools.partial(
        pl.kernel,
        out_type=(
            jax.ShapeDtypeStruct((3, BATCH, HIDDEN), jnp.float32),
            jax.ShapeDtypeStruct((2, BATCH, MSG_DIM), jnp.float32),
        ),
        mesh=plsc.VectorSubcoreMesh(core_axis_name="c", subcore_axis_name="s",
                                    num_cores=NC, num_subcores=NS),
        compiler_params=_SC_PARAMS,
        scratch_types=[
            pltpu.VMEM_SHARED((N_ACC, HIDDEN), jnp.float32),
            pltpu.VMEM_SHARED((N_ACC, MSG_DIM), jnp.float32),
            pltpu.VMEM((CAP,), jnp.int32),
            pltpu.VMEM((CAP,), jnp.int32),
            pltpu.VMEM((CAP,), jnp.int32),
            pltpu.VMEM((CHUNK,), jnp.int32),
            pltpu.VMEM((16,), jnp.int32),
            pltpu.VMEM((B_PER_TILE,), jnp.int32),
            pltpu.VMEM((B_PER_TILE,), jnp.int32),
            pltpu.VMEM((CHUNK, HIDDEN // 2), jnp.int32),
            pltpu.VMEM((CHUNK, HIDDEN // 2), jnp.int32),
            pltpu.VMEM((CHUNK, MSG_DIM), jnp.float32),
            pltpu.VMEM((CHUNK, HIDDEN // 2), jnp.int32),
            pltpu.VMEM((CHUNK, HIDDEN // 2), jnp.int32),
            pltpu.VMEM((CHUNK, MSG_DIM), jnp.float32),
            pltpu.VMEM((CHUNK, HIDDEN), jnp.float32),
            pltpu.VMEM((B_PER_TILE, HIDDEN), jnp.float32),
            pltpu.VMEM((B_PER_TILE, MSG_DIM), jnp.float32),
            pltpu.SemaphoreType.DMA,
            pltpu.SemaphoreType.DMA,
            pltpu.SemaphoreType.DMA,
            pltpu.SemaphoreType.DMA,
            pltpu.SemaphoreType.DMA,
            pltpu.SemaphoreType.DMA,
        ],
    )(_sc_agg_body)


def _cls_body(sel_ref, msel_ref, wm_ref, wu_ref, bu_ref, w1_ref, b1_ref,
              w2_ref, b2_ref, out_ref):
    x = (sel_ref[0] + sel_ref[1] + sel_ref[2]
         + jnp.dot(msel_ref[0] + msel_ref[1], wm_ref[...],
                   preferred_element_type=jnp.float32))
    h = jnp.maximum(
        jnp.dot(x, wu_ref[...], preferred_element_type=jnp.float32)
        + bu_ref[...], 0.0)
    h2 = jnp.maximum(
        jnp.dot(h, w1_ref[...], preferred_element_type=jnp.float32)
        + b1_ref[...], 0.0)
    out_ref[...] = (jnp.dot(h2, w2_ref[...],
                            preferred_element_type=jnp.float32)
                    + b2_ref[...])


def _classifier(sel, msel, W_msg, W_upd, bu, W1, b1, W2, b2):
    return pl.pallas_call(
        _cls_body,
        out_shape=jax.ShapeDtypeStruct((BATCH, HIDDEN), jnp.float32),
    )(sel, msel, W_msg, W_upd, bu, W1, b1, W2, b2)


def kernel(src, dst, t, msg, labels, idx, node_emb, w_time, W_msg, b_msg,
           W_upd, b_upd, W1, b1, W2, b2):
    del labels
    srcc, slotc, eidc, tcc, cnts, slotsel = _make_sc_compact()(
        src, dst, t, idx, jnp.full((N_NODES,), TRASH, jnp.int32))
    F = _cos_features(cnts, tcc, w_time, b_msg.reshape(1, HIDDEN))
    eb = node_emb.astype(jnp.bfloat16)
    lo = (jax.lax.bitcast_convert_type(eb[:, :64], jnp.int16)
          .astype(jnp.int32) & 0xFFFF)
    hi = jax.lax.bitcast_convert_type(eb[:, 64:], jnp.int16).astype(
        jnp.int32) << 16
    emb_sw = hi | lo
    sel, msel = _make_sc_agg()(
        emb_sw, F, msg, node_emb, srcc, slotc, eidc, cnts, slotsel, idx,
        jnp.zeros((N_ACC, HIDDEN), jnp.float32),
        jnp.zeros((N_ACC, MSG_DIM), jnp.float32))
    return _classifier(sel, msel, W_msg, W_upd, b_upd.reshape(1, HIDDEN),
                       W1, b1.reshape(1, HIDDEN), W2, b2.reshape(1, HIDDEN))
